# baseline (speedup 1.0000x reference)
import functools

import jax
import jax.numpy as jnp
from jax import lax
from jax.experimental import pallas as pl
from jax.experimental.pallas import tpu as pltpu
from jax.experimental.pallas import tpu_sc as plsc

N = 50000
D = 128
H = 64
E_POS = 400000
E_NEG = 400000
LAMB = 1.0

BN = 128
NACC = 50176
TROW = NACC // 16
EPAD = 401408
NWIN = EPAD // 128
NW_T = NWIN // 16
NW_L = NWIN // 32
NCORES = 2
NSUB = 16


def _vsmesh():
    return plsc.VectorSubcoreMesh(
        core_axis_name="c", subcore_axis_name="s",
        num_cores=NCORES, num_subcores=NSUB)


def _zero_vbuf(z0):
    def body(i, _):
        z0[i, pl.ds(0, 16)] = jnp.zeros((16,), jnp.float32)
        z0[i, pl.ds(16, 16)] = jnp.zeros((16,), jnp.float32)
        return 0
    lax.fori_loop(0, 128, body, 0)


def _scsum_body(with_cnt_outs, t0, t1, t2, t3, colp, rowp, coln, rown,
                *rest):
    if with_cnt_outs:
        (op0, op1, op2, op3, on0, on1, on2, on3, ocntp, ocntn,
         colbuf, rowbuf, rows_v, z0, z0c, ones_v, cacc, acc,
         gsem, ssem, zsem) = rest
    else:
        (op0, op1, op2, op3, on0, on1, on2, on3,
         colbuf, rowbuf, rows_v, z0, z0c, ones_v, cacc, acc,
         gsem, ssem, zsem) = rest
        ocntp = ocntn = None
    c = lax.axis_index("c")
    s = lax.axis_index("s")

    _zero_vbuf(z0)
    for i in range(8):
        z0c[pl.ds(i * 16, 16)] = jnp.zeros((16,), jnp.float32)
        ones_v[pl.ds(i * 16, 16)] = jnp.ones((16,), jnp.float32)

    def one_pass(t_ref, col2d, row2d, o_ref, ocnt, with_cnt):
        base = s * TROW
        def zb(i, _):
            pltpu.async_copy(z0, acc.at[pl.ds(base + i * 128, 128)], zsem)
            return 0
        lax.fori_loop(0, 24, zb, 0)
        pltpu.async_copy(z0.at[pl.ds(0, 64)],
                         acc.at[pl.ds(base + 24 * 128, 64)], zsem)
        def zbw(i, _):
            pltpu.make_async_copy(
                z0, acc.at[pl.ds(base + i * 128, 128)], zsem).wait()
            return 0
        if with_cnt:
            @pl.when(c == 0)
            def _():
                def zc(i, _):
                    pltpu.sync_copy(z0c, cacc.at[pl.ds(base + i * 128, 128)])
                    return 0
                lax.fori_loop(0, 24, zc, 0)
                pltpu.sync_copy(z0c.at[pl.ds(0, 64)],
                                cacc.at[pl.ds(base + 24 * 128, 64)])
        lax.fori_loop(0, 24, zbw, 0)
        pltpu.make_async_copy(
            z0.at[pl.ds(0, 64)],
            acc.at[pl.ds(base + 24 * 128, 64)], zsem).wait()
        plsc.subcore_barrier()

        for b in (0, 1):
            pltpu.sync_copy(col2d.at[s, b], colbuf.at[b])
            pltpu.sync_copy(row2d.at[s, b], rowbuf.at[b])
            pltpu.async_copy(t_ref.at[colbuf.at[b]], rows_v.at[b],
                             gsem.at[b])

        def win(w, _):
            b = lax.rem(w, 4)
            bg = lax.rem(w + 2, 4)

            @pl.when(w >= 2)
            def _():
                pltpu.make_async_copy(
                    rows_v.at[bg], acc.at[rowbuf.at[bg]], ssem.at[bg]).wait()

            @pl.when(w + 2 < NW_T)
            def _():
                pltpu.sync_copy(col2d.at[s, w + 2], colbuf.at[bg])
                pltpu.sync_copy(row2d.at[s, w + 2], rowbuf.at[bg])
                pltpu.async_copy(t_ref.at[colbuf.at[bg]], rows_v.at[bg],
                                 gsem.at[bg])

            pltpu.make_async_copy(
                t_ref.at[colbuf.at[b]], rows_v.at[b], gsem.at[b]).wait()
            pltpu.async_copy(rows_v.at[b], acc.at[rowbuf.at[b]], ssem.at[b],
                             add=True)
            if with_cnt:
                @pl.when(c == 0)
                def _():
                    pltpu.sync_copy(ones_v, cacc.at[rowbuf.at[b]], add=True)
            return 0
        lax.fori_loop(0, NW_T, win, 0)
        for w in (NW_T - 2, NW_T - 1):
            b = w % 4
            pltpu.make_async_copy(
                rows_v.at[b], acc.at[rowbuf.at[b]], ssem.at[b]).wait()
        plsc.subcore_barrier()

        pltpu.sync_copy(acc.at[pl.ds(base, TROW)], o_ref.at[pl.ds(base, TROW)])
        if with_cnt:
            @pl.when(c == 0)
            def _():
                pltpu.sync_copy(cacc.at[pl.ds(base, TROW)],
                                ocnt.at[pl.ds(base, TROW)])
        plsc.subcore_barrier()

    wc = with_cnt_outs

    @pl.when(c == 0)
    def _():
        one_pass(t0, colp, rowp, op0, ocntp, wc)
        one_pass(t0, coln, rown, on0, ocntn, wc)
        one_pass(t1, colp, rowp, op1, None, False)
        one_pass(t1, coln, rown, on1, None, False)

    @pl.when(c == 1)
    def _():
        one_pass(t2, colp, rowp, op2, None, False)
        one_pass(t2, coln, rown, on2, None, False)
        one_pass(t3, colp, rowp, op3, None, False)
        one_pass(t3, coln, rown, on3, None, False)


def _scsum(t0, t1, t2, t3, colp, rowp, coln, rown, with_cnt):
    f32 = jnp.float32
    sums = [jax.ShapeDtypeStruct((NACC, 32), f32)] * 8
    cnts = [jax.ShapeDtypeStruct((NACC,), f32)] * 2 if with_cnt else []
    kfn = pl.kernel(
        functools.partial(_scsum_body, with_cnt),
        out_type=tuple(sums + cnts),
        mesh=_vsmesh(),
        scratch_types=[
            pltpu.VMEM((4, 128), jnp.int32),
            pltpu.VMEM((4, 128), jnp.int32),
            pltpu.VMEM((4, 128, 32), f32),
            pltpu.VMEM((128, 32), f32),
            pltpu.VMEM((128,), f32),
            pltpu.VMEM((128,), f32),
            pltpu.VMEM_SHARED((NACC,), f32),
            pltpu.VMEM_SHARED((NACC, 32), f32),
            pltpu.SemaphoreType.DMA((4,)),
            pltpu.SemaphoreType.DMA((4,)),
            pltpu.SemaphoreType.DMA,
        ],
        compiler_params=pltpu.CompilerParams(use_tc_tiling_on_sc=False),
    )
    return kfn(t0, t1, t2, t3, colp, rowp, coln, rown)


def _loss_body(z_hbm, p_hbm, ip, jp, kp, inn, jn, kn,
               ologits_p, ologits_n, oterms,
               ibuf, jbuf, kbuf, zi, zj, zk, pi, pj, pk, logbuf, tbuf, gsem):
    c = lax.axis_index("c")
    s = lax.axis_index("s")
    wid = s * NCORES + c

    lanes = lax.iota(jnp.int32, 16)

    def run_set(i2d, j2d, k2d, ologits, trow):
        base_w = wid * NW_L

        def stage(w, b):
            pltpu.sync_copy(i2d.at[wid, w], ibuf.at[b])
            pltpu.sync_copy(j2d.at[wid, w], jbuf.at[b])
            pltpu.sync_copy(k2d.at[wid, w], kbuf.at[b])
            pltpu.async_copy(z_hbm.at[ibuf.at[b]], zi.at[b], gsem.at[b])
            pltpu.async_copy(z_hbm.at[jbuf.at[b]], zj.at[b], gsem.at[b])
            pltpu.async_copy(z_hbm.at[kbuf.at[b]], zk.at[b], gsem.at[b])
            pltpu.async_copy(p_hbm.at[ibuf.at[b]], pi.at[b], gsem.at[b])
            pltpu.async_copy(p_hbm.at[jbuf.at[b]], pj.at[b], gsem.at[b])
            pltpu.async_copy(p_hbm.at[kbuf.at[b]], pk.at[b], gsem.at[b])

        def drain(b):
            pltpu.make_async_copy(z_hbm.at[ibuf.at[b]], zi.at[b],
                                  gsem.at[b]).wait()
            pltpu.make_async_copy(z_hbm.at[jbuf.at[b]], zj.at[b],
                                  gsem.at[b]).wait()
            pltpu.make_async_copy(z_hbm.at[kbuf.at[b]], zk.at[b],
                                  gsem.at[b]).wait()
            pltpu.make_async_copy(p_hbm.at[ibuf.at[b]], pi.at[b],
                                  gsem.at[b]).wait()
            pltpu.make_async_copy(p_hbm.at[jbuf.at[b]], pj.at[b],
                                  gsem.at[b]).wait()
            pltpu.make_async_copy(p_hbm.at[kbuf.at[b]], pk.at[b],
                                  gsem.at[b]).wait()

        stage(0, 0)

        def win(w, tacc):
            b = lax.rem(w, 2)
            bn = lax.rem(w + 1, 2)

            @pl.when(w + 1 < NW_L)
            def _():
                stage(w + 1, bn)

            drain(b)
            zib, zjb, zkb = zi.at[b], zj.at[b], zk.at[b]
            pib, pjb, pkb = pi.at[b], pj.at[b], pk.at[b]
            lgb = logbuf.at[b]

            rot = lanes * 5
            for g in range(8):
                rowv = lanes + (g * 16)

                def feat(f, a):
                    fv = jnp.bitwise_and(rot + f, 127)
                    zif = plsc.load_gather(zib, [rowv, fv])
                    zjf = plsc.load_gather(zjb, [rowv, fv])
                    zkf = plsc.load_gather(zkb, [rowv, fv])
                    return a + zjf * (zkf - zif)
                a = lax.fori_loop(0, 128, feat,
                                  jnp.zeros((16,), jnp.float32), unroll=16)

                c0 = jnp.zeros((16,), jnp.int32)
                sqi = plsc.load_gather(pib, [rowv, c0 + 4])
                sqk = plsc.load_gather(pkb, [rowv, c0 + 4])
                u0 = plsc.load_gather(pib, [rowv, c0])
                u1 = plsc.load_gather(pib, [rowv, c0 + 1])
                v0 = plsc.load_gather(pjb, [rowv, c0 + 2])
                v1 = plsc.load_gather(pjb, [rowv, c0 + 3])
                term = jnp.maximum(sqi - sqk + 2.0 * a, 0.0)
                tacc = tacc + term
                plsc.store_scatter(lgb, [rowv, c0], u0 + v0)
                plsc.store_scatter(lgb, [rowv, c0 + 1], u1 + v1)

            pltpu.sync_copy(lgb,
                            ologits.at[pl.ds((base_w + w) * 128, 128)])
            return tacc

        tacc = lax.fori_loop(0, NW_L, win, jnp.zeros((16,), jnp.float32))
        tbuf[0, pl.ds(0, 16)] = tacc
        pltpu.sync_copy(tbuf, oterms.at[pl.ds(trow + wid, 1)])

    run_set(ip, jp, kp, ologits_p, 0)
    run_set(inn, jn, kn, ologits_n, 32)


def _loss_gather(z, p, ip, jp, kp, inn, jn, kn):
    f32 = jnp.float32
    kfn = pl.kernel(
        _loss_body,
        out_type=(
            jax.ShapeDtypeStruct((EPAD, 2), f32),
            jax.ShapeDtypeStruct((EPAD, 2), f32),
            jax.ShapeDtypeStruct((64, 16), f32),
        ),
        mesh=_vsmesh(),
        scratch_types=[
            pltpu.VMEM((2, 128), jnp.int32),
            pltpu.VMEM((2, 128), jnp.int32),
            pltpu.VMEM((2, 128), jnp.int32),
            pltpu.VMEM((2, 128, 128), f32),
            pltpu.VMEM((2, 128, 128), f32),
            pltpu.VMEM((2, 128, 128), f32),
            pltpu.VMEM((2, 128, 16), f32),
            pltpu.VMEM((2, 128, 16), f32),
            pltpu.VMEM((2, 128, 16), f32),
            pltpu.VMEM((2, 128, 2), f32),
            pltpu.VMEM((1, 16), f32),
            pltpu.SemaphoreType.DMA((2,)),
        ],
        compiler_params=pltpu.CompilerParams(
            use_tc_tiling_on_sc=False, needs_layout_passes=False),
    )
    return kfn(z, p, ip, jp, kp, inn, jn, kn)


def _l2n(x):
    return x / jnp.maximum(
        jnp.sqrt(jnp.sum(x * x, axis=-1, keepdims=True)), 1e-12)


def _layer1_body(x0, x1, x2, x3, sp0, sp1, sp2, sp3, sn0, sn1, sn2, sn3,
                 cp, cn, wp, bp, wn, bn, h0, h1, h2, h3):
    x = jnp.concatenate([x0[...], x1[...], x2[...], x3[...]], axis=1)
    aggp = jnp.concatenate([sp0[...], sp1[...], sp2[...], sp3[...]], axis=1)
    aggn = jnp.concatenate([sn0[...], sn1[...], sn2[...], sn3[...]], axis=1)
    aggp = aggp / jnp.maximum(cp[...], 1.0)
    aggn = aggn / jnp.maximum(cn[...], 1.0)
    hp = jnp.tanh(_l2n(
        jnp.concatenate([aggp, x], axis=1) @ wp[...] + bp[...]))
    hn = jnp.tanh(_l2n(
        jnp.concatenate([aggn, x], axis=1) @ wn[...] + bn[...]))
    h0[...] = hp[:, 0:32]
    h1[...] = hp[:, 32:64]
    h2[...] = hn[:, 0:32]
    h3[...] = hn[:, 32:64]


def _layer1(xq, sp, sn, cp2, cn2, wp, bp, wn, bn):
    f32 = jnp.float32
    bspec = pl.BlockSpec((BN, 32), lambda i: (i, 0))
    cspec = pl.BlockSpec((BN, 1), lambda i: (i, 0))
    full = lambda shp: pl.BlockSpec(shp, lambda i: (0, 0))
    return pl.pallas_call(
        _layer1_body,
        grid=(NACC // BN,),
        in_specs=[bspec] * 12 + [cspec, cspec,
                                 full((2 * D, H)), full((1, H)),
                                 full((2 * D, H)), full((1, H))],
        out_specs=[bspec] * 4,
        out_shape=[jax.ShapeDtypeStruct((NACC, 32), f32)] * 4,
    )(*xq, *sp, *sn, cp2, cn2, wp, bp, wn, bn)


def _layer2_body(h0, h1, h2, h3, pe0, pe1, pe2, pe3, ne0, ne1, ne2, ne3,
                 cp, cn, wp, bp, wn, bn, wr, zo, po):
    hp = jnp.concatenate([h0[...], h1[...]], axis=1)
    hn = jnp.concatenate([h2[...], h3[...]], axis=1)
    icp = 1.0 / jnp.maximum(cp[...], 1.0)
    icn = 1.0 / jnp.maximum(cn[...], 1.0)
    out1 = jnp.concatenate([pe0[...], pe1[...]], axis=1) * icp
    out1n = jnp.concatenate([pe2[...], pe3[...]], axis=1) * icp
    out2n = jnp.concatenate([ne0[...], ne1[...]], axis=1) * icn
    out2 = jnp.concatenate([ne2[...], ne3[...]], axis=1) * icn
    hp2 = jnp.tanh(_l2n(
        jnp.concatenate([out1, out2, hp], axis=1) @ wp[...] + bp[...]))
    hn2 = jnp.tanh(_l2n(
        jnp.concatenate([out1n, out2n, hn], axis=1) @ wn[...] + bn[...]))
    z = jnp.concatenate([hp2, hn2], axis=1)
    zo[...] = z
    wr_full = wr[...]
    u = z @ wr_full[0:128, :]
    v = z @ wr_full[128:256, :]
    sq = jnp.sum(z * z, axis=1, keepdims=True)
    po[...] = jnp.concatenate(
        [u, v, sq, jnp.zeros((z.shape[0], 11), jnp.float32)], axis=1)


def _layer2(hq, spe, sne, cp2, cn2, wp, bp, wn, bn, wr):
    f32 = jnp.float32
    bspec = pl.BlockSpec((BN, 32), lambda i: (i, 0))
    cspec = pl.BlockSpec((BN, 1), lambda i: (i, 0))
    full = lambda shp: pl.BlockSpec(shp, lambda i: (0, 0))
    return pl.pallas_call(
        _layer2_body,
        grid=(NACC // BN,),
        in_specs=[bspec] * 12 + [cspec, cspec,
                                 full((3 * H, H)), full((1, H)),
                                 full((3 * H, H)), full((1, H)),
                                 full((4 * H, 2))],
        out_specs=[pl.BlockSpec((BN, D), lambda i: (i, 0)),
                   pl.BlockSpec((BN, 16), lambda i: (i, 0))],
        out_shape=[jax.ShapeDtypeStruct((NACC, D), f32),
                   jax.ShapeDtypeStruct((NACC, 16), f32)],
    )(*hq, *spe, *sne, cp2, cn2, wp, bp, wn, bn, wr)


def _nll_body(lg, tg, o):
    pid = pl.program_id(0)
    row = jax.lax.broadcasted_iota(jnp.int32, (4096, 1), 0) + pid * 4096
    lgv = lg[...]
    l0 = lgv[:, 0:1]
    l1 = lgv[:, 1:2]
    m = jnp.maximum(l0, l1)
    lse = m + jnp.log(jnp.exp(l0 - m) + jnp.exp(l1 - m))
    lt = jnp.where(tg[...] == 0, l0, l1)
    val = jnp.where(row < E_POS, lse - lt, 0.0)

    @pl.when(pid == 0)
    def _():
        o[...] = jnp.zeros_like(o)
    o[...] += jnp.sum(val, axis=0, keepdims=True)


def _nll_sum(logits, tgt2d):
    return pl.pallas_call(
        _nll_body,
        grid=(EPAD // 4096,),
        in_specs=[pl.BlockSpec((4096, 2), lambda i: (i, 0)),
                  pl.BlockSpec((4096, 1), lambda i: (i, 0))],
        out_specs=pl.BlockSpec((1, 1), lambda i: (0, 0)),
        out_shape=jax.ShapeDtypeStruct((1, 1), jnp.float32),
    )(logits, tgt2d)


def _pad_edges(col, row):
    pad = EPAD - col.shape[0]
    padcol = (jnp.arange(pad, dtype=jnp.int32) * 97) % N
    padrow = N + (jnp.arange(pad, dtype=jnp.int32) % 128)
    col2d = jnp.concatenate([col, padcol]).reshape(16, NW_T, 128)
    row2d = jnp.concatenate([row, padrow]).reshape(16, NW_T, 128)
    return col2d, row2d


def _pad_idx(a):
    pad = EPAD - a.shape[0]
    return jnp.concatenate(
        [a, jnp.zeros((pad,), jnp.int32)]).reshape(32, NW_L, 128)


def kernel(positive_edges, negative_edges, target, pos_samples, neg_samples,
           X, W_pos1, b_pos1, W_neg1, b_neg1, W_pos2, b_pos2, W_neg2, b_neg2,
           W_reg):
    rp, cp = positive_edges[0], positive_edges[1]
    rn, cn = negative_edges[0], negative_edges[1]

    colp2d, rowp3d = _pad_edges(cp, rp)
    coln2d, rown3d = _pad_edges(cn, rn)

    Xp = jnp.pad(X, ((0, NACC - N), (0, 0)))
    xq = tuple(Xp[:, q * 32:(q + 1) * 32] for q in range(4))

    *sums1, cntp, cntn = _scsum(*xq, colp2d, rowp3d, coln2d, rown3d, True)
    sp, sn = sums1[0:4], sums1[4:8]
    cntp = cntp.reshape(NACC, 1)
    cntn = cntn.reshape(NACC, 1)

    hq = _layer1(xq, sp, sn, cntp, cntn,
                 W_pos1, b_pos1.reshape(1, H), W_neg1, b_neg1.reshape(1, H))

    sums2 = _scsum(*hq, colp2d, rowp3d, coln2d, rown3d, False)
    spe, sne = sums2[0:4], sums2[4:8]

    z, ptab = _layer2(hq, spe, sne, cntp, cntn,
                      W_pos2, b_pos2.reshape(1, H),
                      W_neg2, b_neg2.reshape(1, H), W_reg)

    ip2d, jp2d, kp2d = _pad_idx(rp), _pad_idx(cp), _pad_idx(pos_samples)
    in2d, jn2d, kn2d = _pad_idx(rn), _pad_idx(cn), _pad_idx(neg_samples)

    logits_p, logits_n, terms = _loss_gather(
        z, ptab, ip2d, jp2d, kp2d, in2d, jn2d, kn2d)
    terms_p, terms_n = terms[:32], terms[32:]

    tp = jnp.concatenate(
        [target[:E_POS], jnp.zeros((EPAD - E_POS,), jnp.int32)]
    ).reshape(EPAD, 1)
    tn = jnp.concatenate(
        [target[E_POS:], jnp.zeros((EPAD - E_NEG,), jnp.int32)]
    ).reshape(EPAD, 1)
    nll = (_nll_sum(logits_p, tp)[0, 0] + _nll_sum(logits_n, tn)[0, 0]) / (
        E_POS + E_NEG)

    loss_p = jnp.sum(terms_p) / E_POS
    loss_n = jnp.sum(terms_n) / E_NEG
    loss = nll + LAMB * (loss_p + loss_n)
    return (loss, z[:N])


# speedup vs baseline: 1.1623x; 1.1623x over previous
import functools

import jax
import jax.numpy as jnp
from jax import lax
from jax.experimental import pallas as pl
from jax.experimental.pallas import tpu as pltpu
from jax.experimental.pallas import tpu_sc as plsc

N = 50000
D = 128
H = 64
E_POS = 400000
E_NEG = 400000
LAMB = 1.0

BN = 512
NACC = 50176
TROW = NACC // 16
EPAD = 401408
NWIN = EPAD // 128
NW_T = NWIN // 16
NW_L = NWIN // 32
NCORES = 2
NSUB = 16


def _vsmesh():
    return plsc.VectorSubcoreMesh(
        core_axis_name="c", subcore_axis_name="s",
        num_cores=NCORES, num_subcores=NSUB)


def _zero_vbuf(z0):
    def body(i, _):
        z0[i, pl.ds(0, 16)] = jnp.zeros((16,), jnp.float32)
        z0[i, pl.ds(16, 16)] = jnp.zeros((16,), jnp.float32)
        return 0
    lax.fori_loop(0, 128, body, 0)


def _scsum_body(with_cnt_outs, t0, t1, t2, t3, colp, rowp, coln, rown,
                *rest):
    if with_cnt_outs:
        (op0, op1, op2, op3, on0, on1, on2, on3, ocntp, ocntn,
         colbuf, rowbuf, rows_v, z0, z0c, ones_v, cacc, acc,
         gsem, ssem, zsem) = rest
    else:
        (op0, op1, op2, op3, on0, on1, on2, on3,
         colbuf, rowbuf, rows_v, z0, z0c, ones_v, cacc, acc,
         gsem, ssem, zsem) = rest
        ocntp = ocntn = None
    c = lax.axis_index("c")
    s = lax.axis_index("s")

    _zero_vbuf(z0)
    for i in range(8):
        z0c[pl.ds(i * 16, 16)] = jnp.zeros((16,), jnp.float32)
        ones_v[pl.ds(i * 16, 16)] = jnp.ones((16,), jnp.float32)

    def one_pass(t_ref, col2d, row2d, o_ref, ocnt, with_cnt):
        base = s * TROW
        def zb(i, _):
            pltpu.async_copy(z0, acc.at[pl.ds(base + i * 128, 128)], zsem)
            return 0
        lax.fori_loop(0, 24, zb, 0)
        pltpu.async_copy(z0.at[pl.ds(0, 64)],
                         acc.at[pl.ds(base + 24 * 128, 64)], zsem)
        def zbw(i, _):
            pltpu.make_async_copy(
                z0, acc.at[pl.ds(base + i * 128, 128)], zsem).wait()
            return 0
        if with_cnt:
            @pl.when(c == 0)
            def _():
                def zc(i, _):
                    pltpu.sync_copy(z0c, cacc.at[pl.ds(base + i * 128, 128)])
                    return 0
                lax.fori_loop(0, 24, zc, 0)
                pltpu.sync_copy(z0c.at[pl.ds(0, 64)],
                                cacc.at[pl.ds(base + 24 * 128, 64)])
        lax.fori_loop(0, 24, zbw, 0)
        pltpu.make_async_copy(
            z0.at[pl.ds(0, 64)],
            acc.at[pl.ds(base + 24 * 128, 64)], zsem).wait()
        plsc.subcore_barrier()

        ebase = s * (NW_T * 128)
        for b in (0, 1):
            pltpu.sync_copy(col2d.at[pl.ds(ebase + b * 128, 128)],
                            colbuf.at[b])
            pltpu.sync_copy(row2d.at[pl.ds(ebase + b * 128, 128)],
                            rowbuf.at[b])
            pltpu.async_copy(t_ref.at[colbuf.at[b]], rows_v.at[b],
                             gsem.at[b])

        def win(w, _):
            b = lax.rem(w, 4)
            bg = lax.rem(w + 2, 4)

            @pl.when(w >= 2)
            def _():
                pltpu.make_async_copy(
                    rows_v.at[bg], acc.at[rowbuf.at[bg]], ssem.at[bg]).wait()

            @pl.when(w + 2 < NW_T)
            def _():
                pltpu.sync_copy(
                    col2d.at[pl.ds(ebase + (w + 2) * 128, 128)],
                    colbuf.at[bg])
                pltpu.sync_copy(
                    row2d.at[pl.ds(ebase + (w + 2) * 128, 128)],
                    rowbuf.at[bg])
                pltpu.async_copy(t_ref.at[colbuf.at[bg]], rows_v.at[bg],
                                 gsem.at[bg])

            pltpu.make_async_copy(
                t_ref.at[colbuf.at[b]], rows_v.at[b], gsem.at[b]).wait()
            pltpu.async_copy(rows_v.at[b], acc.at[rowbuf.at[b]], ssem.at[b],
                             add=True)
            if with_cnt:
                @pl.when(c == 0)
                def _():
                    pltpu.sync_copy(ones_v, cacc.at[rowbuf.at[b]], add=True)
            return 0
        lax.fori_loop(0, NW_T, win, 0)
        for w in (NW_T - 2, NW_T - 1):
            b = w % 4
            pltpu.make_async_copy(
                rows_v.at[b], acc.at[rowbuf.at[b]], ssem.at[b]).wait()
        plsc.subcore_barrier()

        pltpu.sync_copy(acc.at[pl.ds(base, TROW)], o_ref.at[pl.ds(base, TROW)])
        if with_cnt:
            @pl.when(c == 0)
            def _():
                pltpu.sync_copy(cacc.at[pl.ds(base, TROW)],
                                ocnt.at[pl.ds(base, TROW)])
        plsc.subcore_barrier()

    wc = with_cnt_outs

    @pl.when(c == 0)
    def _():
        one_pass(t0, colp, rowp, op0, ocntp, wc)
        one_pass(t0, coln, rown, on0, ocntn, wc)
        one_pass(t1, colp, rowp, op1, None, False)
        one_pass(t1, coln, rown, on1, None, False)

    @pl.when(c == 1)
    def _():
        one_pass(t2, colp, rowp, op2, None, False)
        one_pass(t2, coln, rown, on2, None, False)
        one_pass(t3, colp, rowp, op3, None, False)
        one_pass(t3, coln, rown, on3, None, False)


def _scsum(t0, t1, t2, t3, colp, rowp, coln, rown, with_cnt):
    f32 = jnp.float32
    sums = [jax.ShapeDtypeStruct((NACC, 32), f32)] * 8
    cnts = [jax.ShapeDtypeStruct((NACC,), f32)] * 2 if with_cnt else []
    kfn = pl.kernel(
        functools.partial(_scsum_body, with_cnt),
        out_type=tuple(sums + cnts),
        mesh=_vsmesh(),
        scratch_types=[
            pltpu.VMEM((4, 128), jnp.int32),
            pltpu.VMEM((4, 128), jnp.int32),
            pltpu.VMEM((4, 128, 32), f32),
            pltpu.VMEM((128, 32), f32),
            pltpu.VMEM((128,), f32),
            pltpu.VMEM((128,), f32),
            pltpu.VMEM_SHARED((NACC,), f32),
            pltpu.VMEM_SHARED((NACC, 32), f32),
            pltpu.SemaphoreType.DMA((4,)),
            pltpu.SemaphoreType.DMA((4,)),
            pltpu.SemaphoreType.DMA,
        ],
        compiler_params=pltpu.CompilerParams(use_tc_tiling_on_sc=False),
    )
    return kfn(t0, t1, t2, t3, colp, rowp, coln, rown)


def _loss_body(z_hbm, p_hbm, ip, jp, kp, inn, jn, kn,
               ologits_p, ologits_n, oterms,
               ibuf, jbuf, kbuf, zi, zj, zk, pi, pj, pk, logbuf, tbuf, gsem):
    c = lax.axis_index("c")
    s = lax.axis_index("s")
    wid = s * NCORES + c

    lanes = lax.iota(jnp.int32, 16)

    def run_set(i2d, j2d, k2d, ologits, trow):
        base_w = wid * NW_L

        def stage(w, b):
            eb = wid * (NW_L * 128)
            pltpu.sync_copy(i2d.at[pl.ds(eb + w * 128, 128)], ibuf.at[b])
            pltpu.sync_copy(j2d.at[pl.ds(eb + w * 128, 128)], jbuf.at[b])
            pltpu.sync_copy(k2d.at[pl.ds(eb + w * 128, 128)], kbuf.at[b])
            pltpu.async_copy(z_hbm.at[ibuf.at[b]], zi.at[b], gsem.at[b])
            pltpu.async_copy(z_hbm.at[jbuf.at[b]], zj.at[b], gsem.at[b])
            pltpu.async_copy(z_hbm.at[kbuf.at[b]], zk.at[b], gsem.at[b])
            pltpu.async_copy(p_hbm.at[ibuf.at[b]], pi.at[b], gsem.at[b])
            pltpu.async_copy(p_hbm.at[jbuf.at[b]], pj.at[b], gsem.at[b])
            pltpu.async_copy(p_hbm.at[kbuf.at[b]], pk.at[b], gsem.at[b])

        def drain(b):
            pltpu.make_async_copy(z_hbm.at[ibuf.at[b]], zi.at[b],
                                  gsem.at[b]).wait()
            pltpu.make_async_copy(z_hbm.at[jbuf.at[b]], zj.at[b],
                                  gsem.at[b]).wait()
            pltpu.make_async_copy(z_hbm.at[kbuf.at[b]], zk.at[b],
                                  gsem.at[b]).wait()
            pltpu.make_async_copy(p_hbm.at[ibuf.at[b]], pi.at[b],
                                  gsem.at[b]).wait()
            pltpu.make_async_copy(p_hbm.at[jbuf.at[b]], pj.at[b],
                                  gsem.at[b]).wait()
            pltpu.make_async_copy(p_hbm.at[kbuf.at[b]], pk.at[b],
                                  gsem.at[b]).wait()

        stage(0, 0)

        def win(w, tacc):
            b = lax.rem(w, 2)
            bn = lax.rem(w + 1, 2)

            @pl.when(w + 1 < NW_L)
            def _():
                stage(w + 1, bn)

            drain(b)
            zib, zjb, zkb = zi.at[b], zj.at[b], zk.at[b]
            pib, pjb, pkb = pi.at[b], pj.at[b], pk.at[b]
            lgb = logbuf.at[b]

            rot = lanes * 5
            for g in range(8):
                rowv = lanes + (g * 16)

                def feat(f, a):
                    fv = jnp.bitwise_and(rot + f, 127)
                    zif = plsc.load_gather(zib, [rowv, fv])
                    zjf = plsc.load_gather(zjb, [rowv, fv])
                    zkf = plsc.load_gather(zkb, [rowv, fv])
                    return a + zjf * (zkf - zif)
                a = lax.fori_loop(0, 128, feat,
                                  jnp.zeros((16,), jnp.float32), unroll=16)

                c0 = jnp.zeros((16,), jnp.int32)
                sqi = plsc.load_gather(pib, [rowv, c0 + 4])
                sqk = plsc.load_gather(pkb, [rowv, c0 + 4])
                u0 = plsc.load_gather(pib, [rowv, c0])
                u1 = plsc.load_gather(pib, [rowv, c0 + 1])
                v0 = plsc.load_gather(pjb, [rowv, c0 + 2])
                v1 = plsc.load_gather(pjb, [rowv, c0 + 3])
                term = jnp.maximum(sqi - sqk + 2.0 * a, 0.0)
                tacc = tacc + term
                plsc.store_scatter(lgb, [rowv, c0], u0 + v0)
                plsc.store_scatter(lgb, [rowv, c0 + 1], u1 + v1)

            pltpu.sync_copy(lgb,
                            ologits.at[pl.ds((base_w + w) * 128, 128)])
            return tacc

        tacc = lax.fori_loop(0, NW_L, win, jnp.zeros((16,), jnp.float32))
        tbuf[0, pl.ds(0, 16)] = tacc
        pltpu.sync_copy(tbuf, oterms.at[pl.ds(trow + wid, 1)])

    run_set(ip, jp, kp, ologits_p, 0)
    run_set(inn, jn, kn, ologits_n, 32)


def _loss_gather(z, p, ip, jp, kp, inn, jn, kn):
    f32 = jnp.float32
    kfn = pl.kernel(
        _loss_body,
        out_type=(
            jax.ShapeDtypeStruct((EPAD, 2), f32),
            jax.ShapeDtypeStruct((EPAD, 2), f32),
            jax.ShapeDtypeStruct((64, 16), f32),
        ),
        mesh=_vsmesh(),
        scratch_types=[
            pltpu.VMEM((2, 128), jnp.int32),
            pltpu.VMEM((2, 128), jnp.int32),
            pltpu.VMEM((2, 128), jnp.int32),
            pltpu.VMEM((2, 128, 128), f32),
            pltpu.VMEM((2, 128, 128), f32),
            pltpu.VMEM((2, 128, 128), f32),
            pltpu.VMEM((2, 128, 16), f32),
            pltpu.VMEM((2, 128, 16), f32),
            pltpu.VMEM((2, 128, 16), f32),
            pltpu.VMEM((2, 128, 2), f32),
            pltpu.VMEM((1, 16), f32),
            pltpu.SemaphoreType.DMA((2,)),
        ],
        compiler_params=pltpu.CompilerParams(
            use_tc_tiling_on_sc=False, needs_layout_passes=False),
    )
    return kfn(z, p, ip, jp, kp, inn, jn, kn)


def _l2n(x):
    return x / jnp.maximum(
        jnp.sqrt(jnp.sum(x * x, axis=-1, keepdims=True)), 1e-12)


def _layer1_body(x0, x1, x2, x3, sp0, sp1, sp2, sp3, sn0, sn1, sn2, sn3,
                 cp, cn, wp, bp, wn, bn, h0, h1, h2, h3):
    x = jnp.concatenate([x0[...], x1[...], x2[...], x3[...]], axis=1)
    aggp = jnp.concatenate([sp0[...], sp1[...], sp2[...], sp3[...]], axis=1)
    aggn = jnp.concatenate([sn0[...], sn1[...], sn2[...], sn3[...]], axis=1)
    aggp = aggp / jnp.maximum(cp[...], 1.0)
    aggn = aggn / jnp.maximum(cn[...], 1.0)
    hp = jnp.tanh(_l2n(
        jnp.concatenate([aggp, x], axis=1) @ wp[...] + bp[...]))
    hn = jnp.tanh(_l2n(
        jnp.concatenate([aggn, x], axis=1) @ wn[...] + bn[...]))
    h0[...] = hp[:, 0:32]
    h1[...] = hp[:, 32:64]
    h2[...] = hn[:, 0:32]
    h3[...] = hn[:, 32:64]


def _layer1(xq, sp, sn, cp2, cn2, wp, bp, wn, bn):
    f32 = jnp.float32
    bspec = pl.BlockSpec((BN, 32), lambda i: (i, 0))
    cspec = pl.BlockSpec((BN, 1), lambda i: (i, 0))
    full = lambda shp: pl.BlockSpec(shp, lambda i: (0, 0))
    return pl.pallas_call(
        _layer1_body,
        grid=(NACC // BN,),
        in_specs=[bspec] * 12 + [cspec, cspec,
                                 full((2 * D, H)), full((1, H)),
                                 full((2 * D, H)), full((1, H))],
        out_specs=[bspec] * 4,
        out_shape=[jax.ShapeDtypeStruct((NACC, 32), f32)] * 4,
    )(*xq, *sp, *sn, cp2, cn2, wp, bp, wn, bn)


def _layer2_body(h0, h1, h2, h3, pe0, pe1, pe2, pe3, ne0, ne1, ne2, ne3,
                 cp, cn, wp, bp, wn, bn, wr, zo, po):
    hp = jnp.concatenate([h0[...], h1[...]], axis=1)
    hn = jnp.concatenate([h2[...], h3[...]], axis=1)
    icp = 1.0 / jnp.maximum(cp[...], 1.0)
    icn = 1.0 / jnp.maximum(cn[...], 1.0)
    out1 = jnp.concatenate([pe0[...], pe1[...]], axis=1) * icp
    out1n = jnp.concatenate([pe2[...], pe3[...]], axis=1) * icp
    out2n = jnp.concatenate([ne0[...], ne1[...]], axis=1) * icn
    out2 = jnp.concatenate([ne2[...], ne3[...]], axis=1) * icn
    hp2 = jnp.tanh(_l2n(
        jnp.concatenate([out1, out2, hp], axis=1) @ wp[...] + bp[...]))
    hn2 = jnp.tanh(_l2n(
        jnp.concatenate([out1n, out2n, hn], axis=1) @ wn[...] + bn[...]))
    z = jnp.concatenate([hp2, hn2], axis=1)
    zo[...] = z
    wr_full = wr[...]
    u = z @ wr_full[0:128, :]
    v = z @ wr_full[128:256, :]
    sq = jnp.sum(z * z, axis=1, keepdims=True)
    po[...] = jnp.concatenate(
        [u, v, sq, jnp.zeros((z.shape[0], 11), jnp.float32)], axis=1)


def _layer2(hq, spe, sne, cp2, cn2, wp, bp, wn, bn, wr):
    f32 = jnp.float32
    bspec = pl.BlockSpec((BN, 32), lambda i: (i, 0))
    cspec = pl.BlockSpec((BN, 1), lambda i: (i, 0))
    full = lambda shp: pl.BlockSpec(shp, lambda i: (0, 0))
    return pl.pallas_call(
        _layer2_body,
        grid=(NACC // BN,),
        in_specs=[bspec] * 12 + [cspec, cspec,
                                 full((3 * H, H)), full((1, H)),
                                 full((3 * H, H)), full((1, H)),
                                 full((4 * H, 2))],
        out_specs=[pl.BlockSpec((BN, D), lambda i: (i, 0)),
                   pl.BlockSpec((BN, 16), lambda i: (i, 0))],
        out_shape=[jax.ShapeDtypeStruct((NACC, D), f32),
                   jax.ShapeDtypeStruct((NACC, 16), f32)],
    )(*hq, *spe, *sne, cp2, cn2, wp, bp, wn, bn, wr)


NLR = EPAD * 2 // 128
BE = NLR // 7


def _nll_body(lp, tp, ln, tn, o):
    pid = pl.program_id(0)

    def one(lg, tg):
        x = lg[...]
        t = tg[...]
        b = jnp.roll(x, -1, axis=1)
        m = jnp.maximum(x, b)
        lse = m + jnp.log(jnp.exp(x - m) + jnp.exp(b - m))
        lt = jnp.where(t == 0, x, b)
        lane = jax.lax.broadcasted_iota(jnp.int32, (BE, 128), 1)
        row = jax.lax.broadcasted_iota(jnp.int32, (BE, 128), 0) + pid * BE
        edge = row * 64 + lane // 2
        mask = (jnp.bitwise_and(lane, 1) == 0) & (edge < E_POS)
        return jnp.sum(jnp.where(mask, lse - lt, 0.0))

    @pl.when(pid == 0)
    def _():
        o[...] = jnp.zeros_like(o)
    o[...] += (one(lp, tp) + one(ln, tn)).reshape(1, 1)


def _nll_sum(logits_p, tp2, logits_n, tn2):
    bspec = pl.BlockSpec((BE, 128), lambda i: (i, 0))
    return pl.pallas_call(
        _nll_body,
        grid=(7,),
        in_specs=[bspec] * 4,
        out_specs=pl.BlockSpec((1, 1), lambda i: (0, 0)),
        out_shape=jax.ShapeDtypeStruct((1, 1), jnp.float32),
    )(logits_p, tp2, logits_n, tn2)


def _pad_edges(col, row):
    pad = EPAD - col.shape[0]
    padcol = (jnp.arange(pad, dtype=jnp.int32) * 97) % N
    padrow = N + (jnp.arange(pad, dtype=jnp.int32) % 128)
    return jnp.concatenate([col, padcol]), jnp.concatenate([row, padrow])


def _pad_idx(a):
    pad = EPAD - a.shape[0]
    return jnp.concatenate([a, jnp.zeros((pad,), jnp.int32)])


def kernel(positive_edges, negative_edges, target, pos_samples, neg_samples,
           X, W_pos1, b_pos1, W_neg1, b_neg1, W_pos2, b_pos2, W_neg2, b_neg2,
           W_reg):
    rp, cp = positive_edges[0], positive_edges[1]
    rn, cn = negative_edges[0], negative_edges[1]

    colp2d, rowp3d = _pad_edges(cp, rp)
    coln2d, rown3d = _pad_edges(cn, rn)

    Xp = jnp.pad(X, ((0, NACC - N), (0, 0)))
    xq = tuple(Xp[:, q * 32:(q + 1) * 32] for q in range(4))

    *sums1, cntp, cntn = _scsum(*xq, colp2d, rowp3d, coln2d, rown3d, True)
    sp, sn = sums1[0:4], sums1[4:8]
    cntp = cntp.reshape(NACC, 1)
    cntn = cntn.reshape(NACC, 1)

    hq = _layer1(xq, sp, sn, cntp, cntn,
                 W_pos1, b_pos1.reshape(1, H), W_neg1, b_neg1.reshape(1, H))

    sums2 = _scsum(*hq, colp2d, rowp3d, coln2d, rown3d, False)
    spe, sne = sums2[0:4], sums2[4:8]

    z, ptab = _layer2(hq, spe, sne, cntp, cntn,
                      W_pos2, b_pos2.reshape(1, H),
                      W_neg2, b_neg2.reshape(1, H), W_reg)

    ip2d, jp2d, kp2d = _pad_idx(rp), _pad_idx(cp), _pad_idx(pos_samples)
    in2d, jn2d, kn2d = _pad_idx(rn), _pad_idx(cn), _pad_idx(neg_samples)

    logits_p, logits_n, terms = _loss_gather(
        z, ptab, ip2d, jp2d, kp2d, in2d, jn2d, kn2d)
    terms_p, terms_n = terms[:32], terms[32:]

    tp2 = jnp.repeat(
        jnp.concatenate([target[:E_POS], jnp.zeros((EPAD - E_POS,),
                                                   jnp.int32)]), 2
    ).reshape(NLR, 128)
    tn2 = jnp.repeat(
        jnp.concatenate([target[E_POS:], jnp.zeros((EPAD - E_NEG,),
                                                   jnp.int32)]), 2
    ).reshape(NLR, 128)
    nll = _nll_sum(logits_p.reshape(NLR, 128), tp2,
                   logits_n.reshape(NLR, 128), tn2)[0, 0] / (E_POS + E_NEG)

    loss_p = jnp.sum(terms_p) / E_POS
    loss_n = jnp.sum(terms_n) / E_NEG
    loss = nll + LAMB * (loss_p + loss_n)
    return (loss, z[:N])


# speedup vs baseline: 1.5832x; 1.3621x over previous
import functools

import jax
import jax.numpy as jnp
from jax import lax
from jax.experimental import pallas as pl
from jax.experimental.pallas import tpu as pltpu
from jax.experimental.pallas import tpu_sc as plsc

N = 50000
D = 128
H = 64
E_POS = 400000
E_NEG = 400000
LAMB = 1.0

BN = 512
NACC = 50176
TROW = NACC // 16
EPAD = 401408
NWIN = EPAD // 128
NW_T = NWIN // 16
NW_L = NWIN // 32
NCORES = 2
NSUB = 16


def _vsmesh():
    return plsc.VectorSubcoreMesh(
        core_axis_name="c", subcore_axis_name="s",
        num_cores=NCORES, num_subcores=NSUB)


def _zero_vbuf(z0):
    def body(i, _):
        z0[i, pl.ds(0, 16)] = jnp.zeros((16,), jnp.float32)
        z0[i, pl.ds(16, 16)] = jnp.zeros((16,), jnp.float32)
        return 0
    lax.fori_loop(0, 128, body, 0)


def _scsum_body(with_cnt_outs, t0, t1, t2, t3, colp, rowp, coln, rown,
                *rest):
    if with_cnt_outs:
        (op0, op1, op2, op3, on0, on1, on2, on3, ocntp, ocntn,
         colbuf, rowbuf, rows_v, z0, z0c, ones_v, cacc, acc,
         gsem, ssem, isem, zsem) = rest
    else:
        (op0, op1, op2, op3, on0, on1, on2, on3,
         colbuf, rowbuf, rows_v, z0, z0c, ones_v, cacc, acc,
         gsem, ssem, isem, zsem) = rest
        ocntp = ocntn = None
    c = lax.axis_index("c")
    s = lax.axis_index("s")

    _zero_vbuf(z0)
    for i in range(8):
        z0c[pl.ds(i * 16, 16)] = jnp.zeros((16,), jnp.float32)
        ones_v[pl.ds(i * 16, 16)] = jnp.ones((16,), jnp.float32)

    def one_pass(t_ref, col2d, row2d, o_ref, ocnt, with_cnt):
        base = s * TROW
        def zb(i, _):
            pltpu.async_copy(z0, acc.at[pl.ds(base + i * 128, 128)], zsem)
            return 0
        lax.fori_loop(0, 24, zb, 0)
        pltpu.async_copy(z0.at[pl.ds(0, 64)],
                         acc.at[pl.ds(base + 24 * 128, 64)], zsem)
        def zbw(i, _):
            pltpu.make_async_copy(
                z0, acc.at[pl.ds(base + i * 128, 128)], zsem).wait()
            return 0
        if with_cnt:
            @pl.when(c == 0)
            def _():
                def zc(i, _):
                    pltpu.sync_copy(z0c, cacc.at[pl.ds(base + i * 128, 128)])
                    return 0
                lax.fori_loop(0, 24, zc, 0)
                pltpu.sync_copy(z0c.at[pl.ds(0, 64)],
                                cacc.at[pl.ds(base + 24 * 128, 64)])
        lax.fori_loop(0, 24, zbw, 0)
        pltpu.make_async_copy(
            z0.at[pl.ds(0, 64)],
            acc.at[pl.ds(base + 24 * 128, 64)], zsem).wait()
        plsc.subcore_barrier()

        ebase = s * (NW_T * 128)

        def stage_idx(w):
            b8 = lax.rem(w, 8)
            pltpu.async_copy(col2d.at[pl.ds(ebase + w * 128, 128)],
                             colbuf.at[b8], isem.at[b8])
            pltpu.async_copy(row2d.at[pl.ds(ebase + w * 128, 128)],
                             rowbuf.at[b8], isem.at[b8])

        def wait_idx(w):
            b8 = lax.rem(w, 8)
            pltpu.make_async_copy(
                col2d.at[pl.ds(ebase + w * 128, 128)],
                colbuf.at[b8], isem.at[b8]).wait()
            pltpu.make_async_copy(
                row2d.at[pl.ds(ebase + w * 128, 128)],
                rowbuf.at[b8], isem.at[b8]).wait()

        for w0 in (0, 1, 2):
            stage_idx(w0)
        for w0 in (0, 1):
            wait_idx(w0)
            pltpu.async_copy(t_ref.at[colbuf.at[w0]], rows_v.at[w0],
                             gsem.at[w0])

        def win(w, _):
            b = lax.rem(w, 4)
            b8 = lax.rem(w, 8)
            bg = lax.rem(w + 2, 4)
            bg8 = lax.rem(w + 2, 8)

            @pl.when(w >= 2)
            def _():
                pltpu.make_async_copy(
                    rows_v.at[bg], acc.at[rowbuf.at[lax.rem(w - 2, 8)]],
                    ssem.at[bg]).wait()

            @pl.when(w + 3 < NW_T)
            def _():
                stage_idx(w + 3)

            @pl.when(w + 2 < NW_T)
            def _():
                wait_idx(w + 2)
                pltpu.async_copy(t_ref.at[colbuf.at[bg8]], rows_v.at[bg],
                                 gsem.at[bg])

            pltpu.make_async_copy(
                t_ref.at[colbuf.at[b8]], rows_v.at[b], gsem.at[b]).wait()
            pltpu.async_copy(rows_v.at[b], acc.at[rowbuf.at[b8]],
                             ssem.at[b], add=True)
            if with_cnt:
                @pl.when(c == 0)
                def _():
                    pltpu.sync_copy(ones_v, cacc.at[rowbuf.at[b8]], add=True)
            return 0
        lax.fori_loop(0, NW_T, win, 0)
        for w in (NW_T - 2, NW_T - 1):
            pltpu.make_async_copy(
                rows_v.at[w % 4], acc.at[rowbuf.at[w % 8]],
                ssem.at[w % 4]).wait()
        plsc.subcore_barrier()

        pltpu.sync_copy(acc.at[pl.ds(base, TROW)], o_ref.at[pl.ds(base, TROW)])
        if with_cnt:
            @pl.when(c == 0)
            def _():
                pltpu.sync_copy(cacc.at[pl.ds(base, TROW)],
                                ocnt.at[pl.ds(base, TROW)])
        plsc.subcore_barrier()

    wc = with_cnt_outs

    @pl.when(c == 0)
    def _():
        one_pass(t0, colp, rowp, op0, ocntp, wc)
        one_pass(t0, coln, rown, on0, ocntn, wc)
        one_pass(t1, colp, rowp, op1, None, False)
        one_pass(t1, coln, rown, on1, None, False)

    @pl.when(c == 1)
    def _():
        one_pass(t2, colp, rowp, op2, None, False)
        one_pass(t2, coln, rown, on2, None, False)
        one_pass(t3, colp, rowp, op3, None, False)
        one_pass(t3, coln, rown, on3, None, False)


def _scsum(t0, t1, t2, t3, colp, rowp, coln, rown, with_cnt):
    f32 = jnp.float32
    sums = [jax.ShapeDtypeStruct((NACC, 32), f32)] * 8
    cnts = [jax.ShapeDtypeStruct((NACC,), f32)] * 2 if with_cnt else []
    kfn = pl.kernel(
        functools.partial(_scsum_body, with_cnt),
        out_type=tuple(sums + cnts),
        mesh=_vsmesh(),
        scratch_types=[
            pltpu.VMEM((8, 128), jnp.int32),
            pltpu.VMEM((8, 128), jnp.int32),
            pltpu.VMEM((4, 128, 32), f32),
            pltpu.VMEM((128, 32), f32),
            pltpu.VMEM((128,), f32),
            pltpu.VMEM((128,), f32),
            pltpu.VMEM_SHARED((NACC,), f32),
            pltpu.VMEM_SHARED((NACC, 32), f32),
            pltpu.SemaphoreType.DMA((4,)),
            pltpu.SemaphoreType.DMA((4,)),
            pltpu.SemaphoreType.DMA((8,)),
            pltpu.SemaphoreType.DMA,
        ],
        compiler_params=pltpu.CompilerParams(use_tc_tiling_on_sc=False),
    )
    return kfn(t0, t1, t2, t3, colp, rowp, coln, rown)


def _loss_body(z_hbm, p_hbm, ip, jp, kp, inn, jn, kn,
               ologits_p, ologits_n, oterms,
               ibuf, jbuf, kbuf, zi, zj, zk, pi, pj, pk, logbuf, tbuf, gsem):
    c = lax.axis_index("c")
    s = lax.axis_index("s")
    wid = s * NCORES + c

    lanes = lax.iota(jnp.int32, 16)

    def run_set(i2d, j2d, k2d, ologits, trow):
        base_w = wid * NW_L

        def stage(w, b):
            eb = wid * (NW_L * 128)
            pltpu.sync_copy(i2d.at[pl.ds(eb + w * 128, 128)], ibuf.at[b])
            pltpu.sync_copy(j2d.at[pl.ds(eb + w * 128, 128)], jbuf.at[b])
            pltpu.sync_copy(k2d.at[pl.ds(eb + w * 128, 128)], kbuf.at[b])
            pltpu.async_copy(z_hbm.at[ibuf.at[b]], zi.at[b], gsem.at[b])
            pltpu.async_copy(z_hbm.at[jbuf.at[b]], zj.at[b], gsem.at[b])
            pltpu.async_copy(z_hbm.at[kbuf.at[b]], zk.at[b], gsem.at[b])
            pltpu.async_copy(p_hbm.at[ibuf.at[b]], pi.at[b], gsem.at[b])
            pltpu.async_copy(p_hbm.at[jbuf.at[b]], pj.at[b], gsem.at[b])
            pltpu.async_copy(p_hbm.at[kbuf.at[b]], pk.at[b], gsem.at[b])

        def drain(b):
            pltpu.make_async_copy(z_hbm.at[ibuf.at[b]], zi.at[b],
                                  gsem.at[b]).wait()
            pltpu.make_async_copy(z_hbm.at[jbuf.at[b]], zj.at[b],
                                  gsem.at[b]).wait()
            pltpu.make_async_copy(z_hbm.at[kbuf.at[b]], zk.at[b],
                                  gsem.at[b]).wait()
            pltpu.make_async_copy(p_hbm.at[ibuf.at[b]], pi.at[b],
                                  gsem.at[b]).wait()
            pltpu.make_async_copy(p_hbm.at[jbuf.at[b]], pj.at[b],
                                  gsem.at[b]).wait()
            pltpu.make_async_copy(p_hbm.at[kbuf.at[b]], pk.at[b],
                                  gsem.at[b]).wait()

        stage(0, 0)

        def win(w, tacc):
            b = lax.rem(w, 2)
            bn = lax.rem(w + 1, 2)

            @pl.when(w + 1 < NW_L)
            def _():
                stage(w + 1, bn)

            drain(b)
            zib, zjb, zkb = zi.at[b], zj.at[b], zk.at[b]
            pib, pjb, pkb = pi.at[b], pj.at[b], pk.at[b]
            lgb = logbuf.at[b]

            rot = lanes * 5
            for g in range(8):
                rowv = lanes + (g * 16)

                def feat(f, a):
                    fv = jnp.bitwise_and(rot + f, 127)
                    zif = plsc.load_gather(zib, [rowv, fv])
                    zjf = plsc.load_gather(zjb, [rowv, fv])
                    zkf = plsc.load_gather(zkb, [rowv, fv])
                    return a + zjf * (zkf - zif)
                a = lax.fori_loop(0, 128, feat,
                                  jnp.zeros((16,), jnp.float32), unroll=16)

                c0 = jnp.zeros((16,), jnp.int32)
                sqi = plsc.load_gather(pib, [rowv, c0 + 4])
                sqk = plsc.load_gather(pkb, [rowv, c0 + 4])
                u0 = plsc.load_gather(pib, [rowv, c0])
                u1 = plsc.load_gather(pib, [rowv, c0 + 1])
                v0 = plsc.load_gather(pjb, [rowv, c0 + 2])
                v1 = plsc.load_gather(pjb, [rowv, c0 + 3])
                term = jnp.maximum(sqi - sqk + 2.0 * a, 0.0)
                tacc = tacc + term
                rr = jnp.zeros((16,), jnp.int32) + (g // 4)
                cc = jnp.bitwise_and(rowv * 2, 127)
                plsc.store_scatter(lgb, [rr, cc], u0 + v0)
                plsc.store_scatter(lgb, [rr, cc + 1], u1 + v1)

            pltpu.sync_copy(lgb,
                            ologits.at[pl.ds((base_w + w) * 2, 2)])
            return tacc

        tacc = lax.fori_loop(0, NW_L, win, jnp.zeros((16,), jnp.float32))
        tbuf[0, pl.ds(0, 16)] = tacc
        pltpu.sync_copy(tbuf, oterms.at[pl.ds(trow + wid, 1)])

    run_set(ip, jp, kp, ologits_p, 0)
    run_set(inn, jn, kn, ologits_n, 32)


def _loss_gather(z, p, ip, jp, kp, inn, jn, kn):
    f32 = jnp.float32
    kfn = pl.kernel(
        _loss_body,
        out_type=(
            jax.ShapeDtypeStruct((NLR, 128), f32),
            jax.ShapeDtypeStruct((NLR, 128), f32),
            jax.ShapeDtypeStruct((64, 16), f32),
        ),
        mesh=_vsmesh(),
        scratch_types=[
            pltpu.VMEM((2, 128), jnp.int32),
            pltpu.VMEM((2, 128), jnp.int32),
            pltpu.VMEM((2, 128), jnp.int32),
            pltpu.VMEM((2, 128, 128), f32),
            pltpu.VMEM((2, 128, 128), f32),
            pltpu.VMEM((2, 128, 128), f32),
            pltpu.VMEM((2, 128, 16), f32),
            pltpu.VMEM((2, 128, 16), f32),
            pltpu.VMEM((2, 128, 16), f32),
            pltpu.VMEM((2, 2, 128), f32),
            pltpu.VMEM((1, 16), f32),
            pltpu.SemaphoreType.DMA((2,)),
        ],
        compiler_params=pltpu.CompilerParams(
            use_tc_tiling_on_sc=False, needs_layout_passes=False),
    )
    return kfn(z, p, ip, jp, kp, inn, jn, kn)


def _l2n(x):
    return x / jnp.maximum(
        jnp.sqrt(jnp.sum(x * x, axis=-1, keepdims=True)), 1e-12)


def _layer1_body(x0, x1, x2, x3, sp0, sp1, sp2, sp3, sn0, sn1, sn2, sn3,
                 cp, cn, wp, bp, wn, bn, h0, h1, h2, h3):
    x = jnp.concatenate([x0[...], x1[...], x2[...], x3[...]], axis=1)
    aggp = jnp.concatenate([sp0[...], sp1[...], sp2[...], sp3[...]], axis=1)
    aggn = jnp.concatenate([sn0[...], sn1[...], sn2[...], sn3[...]], axis=1)
    aggp = aggp / jnp.maximum(cp[...], 1.0)
    aggn = aggn / jnp.maximum(cn[...], 1.0)
    hp = jnp.tanh(_l2n(
        jnp.concatenate([aggp, x], axis=1) @ wp[...] + bp[...]))
    hn = jnp.tanh(_l2n(
        jnp.concatenate([aggn, x], axis=1) @ wn[...] + bn[...]))
    h0[...] = hp[:, 0:32]
    h1[...] = hp[:, 32:64]
    h2[...] = hn[:, 0:32]
    h3[...] = hn[:, 32:64]


def _layer1(xq, sp, sn, cp2, cn2, wp, bp, wn, bn):
    f32 = jnp.float32
    bspec = pl.BlockSpec((BN, 32), lambda i: (i, 0))
    cspec = pl.BlockSpec((BN, 1), lambda i: (i, 0))
    full = lambda shp: pl.BlockSpec(shp, lambda i: (0, 0))
    return pl.pallas_call(
        _layer1_body,
        grid=(NACC // BN,),
        in_specs=[bspec] * 12 + [cspec, cspec,
                                 full((2 * D, H)), full((1, H)),
                                 full((2 * D, H)), full((1, H))],
        out_specs=[bspec] * 4,
        out_shape=[jax.ShapeDtypeStruct((NACC, 32), f32)] * 4,
    )(*xq, *sp, *sn, cp2, cn2, wp, bp, wn, bn)


def _layer2_body(h0, h1, h2, h3, pe0, pe1, pe2, pe3, ne0, ne1, ne2, ne3,
                 cp, cn, wp, bp, wn, bn, wr, zo, po):
    hp = jnp.concatenate([h0[...], h1[...]], axis=1)
    hn = jnp.concatenate([h2[...], h3[...]], axis=1)
    icp = 1.0 / jnp.maximum(cp[...], 1.0)
    icn = 1.0 / jnp.maximum(cn[...], 1.0)
    out1 = jnp.concatenate([pe0[...], pe1[...]], axis=1) * icp
    out1n = jnp.concatenate([pe2[...], pe3[...]], axis=1) * icp
    out2n = jnp.concatenate([ne0[...], ne1[...]], axis=1) * icn
    out2 = jnp.concatenate([ne2[...], ne3[...]], axis=1) * icn
    hp2 = jnp.tanh(_l2n(
        jnp.concatenate([out1, out2, hp], axis=1) @ wp[...] + bp[...]))
    hn2 = jnp.tanh(_l2n(
        jnp.concatenate([out1n, out2n, hn], axis=1) @ wn[...] + bn[...]))
    z = jnp.concatenate([hp2, hn2], axis=1)
    zo[...] = z
    wr_full = wr[...]
    u = z @ wr_full[0:128, :]
    v = z @ wr_full[128:256, :]
    sq = jnp.sum(z * z, axis=1, keepdims=True)
    po[...] = jnp.concatenate(
        [u, v, sq, jnp.zeros((z.shape[0], 11), jnp.float32)], axis=1)


def _layer2(hq, spe, sne, cp2, cn2, wp, bp, wn, bn, wr):
    f32 = jnp.float32
    bspec = pl.BlockSpec((BN, 32), lambda i: (i, 0))
    cspec = pl.BlockSpec((BN, 1), lambda i: (i, 0))
    full = lambda shp: pl.BlockSpec(shp, lambda i: (0, 0))
    return pl.pallas_call(
        _layer2_body,
        grid=(NACC // BN,),
        in_specs=[bspec] * 12 + [cspec, cspec,
                                 full((3 * H, H)), full((1, H)),
                                 full((3 * H, H)), full((1, H)),
                                 full((4 * H, 2))],
        out_specs=[pl.BlockSpec((BN, D), lambda i: (i, 0)),
                   pl.BlockSpec((BN, 16), lambda i: (i, 0))],
        out_shape=[jax.ShapeDtypeStruct((NACC, D), f32),
                   jax.ShapeDtypeStruct((NACC, 16), f32)],
    )(*hq, *spe, *sne, cp2, cn2, wp, bp, wn, bn, wr)


NLR = EPAD * 2 // 128
BE = NLR // 7


def _nll_body(lp, tp, ln, tn, o):
    pid = pl.program_id(0)

    def one(lg, tg):
        x = lg[...]
        t = jnp.repeat(tg[...], 2, axis=1)
        b = jnp.roll(x, -1, axis=1)
        m = jnp.maximum(x, b)
        lse = m + jnp.log(jnp.exp(x - m) + jnp.exp(b - m))
        lt = jnp.where(t == 0, x, b)
        lane = jax.lax.broadcasted_iota(jnp.int32, (BE, 128), 1)
        row = jax.lax.broadcasted_iota(jnp.int32, (BE, 128), 0) + pid * BE
        edge = row * 64 + lane // 2
        mask = (jnp.bitwise_and(lane, 1) == 0) & (edge < E_POS)
        return jnp.sum(jnp.where(mask, lse - lt, 0.0))

    @pl.when(pid == 0)
    def _():
        o[...] = jnp.zeros_like(o)
    o[...] += (one(lp, tp) + one(ln, tn)).reshape(1, 1)


def _nll_sum(logits_p, tp2, logits_n, tn2):
    bspec = pl.BlockSpec((BE, 128), lambda i: (i, 0))
    tspec = pl.BlockSpec((BE, 64), lambda i: (i, 0))
    return pl.pallas_call(
        _nll_body,
        grid=(7,),
        in_specs=[bspec, tspec, bspec, tspec],
        out_specs=pl.BlockSpec((1, 1), lambda i: (0, 0)),
        out_shape=jax.ShapeDtypeStruct((1, 1), jnp.float32),
    )(logits_p, tp2, logits_n, tn2)


def _pad_edges(col, row):
    pad = EPAD - col.shape[0]
    padcol = (jnp.arange(pad, dtype=jnp.int32) * 97) % N
    padrow = N + (jnp.arange(pad, dtype=jnp.int32) % 128)
    return jnp.concatenate([col, padcol]), jnp.concatenate([row, padrow])


def _pad_idx(a):
    pad = EPAD - a.shape[0]
    return jnp.concatenate([a, jnp.zeros((pad,), jnp.int32)])


def kernel(positive_edges, negative_edges, target, pos_samples, neg_samples,
           X, W_pos1, b_pos1, W_neg1, b_neg1, W_pos2, b_pos2, W_neg2, b_neg2,
           W_reg):
    rp, cp = positive_edges[0], positive_edges[1]
    rn, cn = negative_edges[0], negative_edges[1]

    colp2d, rowp3d = _pad_edges(cp, rp)
    coln2d, rown3d = _pad_edges(cn, rn)

    Xp = jnp.pad(X, ((0, NACC - N), (0, 0)))
    xq = tuple(Xp[:, q * 32:(q + 1) * 32] for q in range(4))

    *sums1, cntp, cntn = _scsum(*xq, colp2d, rowp3d, coln2d, rown3d, True)
    sp, sn = sums1[0:4], sums1[4:8]
    cntp = cntp.reshape(NACC, 1)
    cntn = cntn.reshape(NACC, 1)

    hq = _layer1(xq, sp, sn, cntp, cntn,
                 W_pos1, b_pos1.reshape(1, H), W_neg1, b_neg1.reshape(1, H))

    sums2 = _scsum(*hq, colp2d, rowp3d, coln2d, rown3d, False)
    spe, sne = sums2[0:4], sums2[4:8]

    z, ptab = _layer2(hq, spe, sne, cntp, cntn,
                      W_pos2, b_pos2.reshape(1, H),
                      W_neg2, b_neg2.reshape(1, H), W_reg)

    ip2d, jp2d, kp2d = _pad_idx(rp), _pad_idx(cp), _pad_idx(pos_samples)
    in2d, jn2d, kn2d = _pad_idx(rn), _pad_idx(cn), _pad_idx(neg_samples)

    logits_p, logits_n, terms = _loss_gather(
        z, ptab, ip2d, jp2d, kp2d, in2d, jn2d, kn2d)
    terms_p, terms_n = terms[:32], terms[32:]

    tp2 = jnp.concatenate(
        [target[:E_POS], jnp.zeros((EPAD - E_POS,), jnp.int32)]
    ).reshape(NLR, 64)
    tn2 = jnp.concatenate(
        [target[E_POS:], jnp.zeros((EPAD - E_NEG,), jnp.int32)]
    ).reshape(NLR, 64)
    nll = _nll_sum(logits_p, tp2,
                   logits_n, tn2)[0, 0] / (E_POS + E_NEG)

    loss_p = jnp.sum(terms_p) / E_POS
    loss_n = jnp.sum(terms_n) / E_NEG
    loss = nll + LAMB * (loss_p + loss_n)
    return (loss, z[:N])


# speedup vs baseline: 1.6994x; 1.0734x over previous
import functools

import jax
import jax.numpy as jnp
from jax import lax
from jax.experimental import pallas as pl
from jax.experimental.pallas import tpu as pltpu
from jax.experimental.pallas import tpu_sc as plsc

N = 50000
D = 128
H = 64
E_POS = 400000
E_NEG = 400000
LAMB = 1.0

BN = 512
NACC = 50176
TROW = NACC // 16
EPAD = 401408
NWIN = EPAD // 128
NW_T = NWIN // 16
NW_L = NWIN // 32
NCORES = 2
NSUB = 16


def _vsmesh():
    return plsc.VectorSubcoreMesh(
        core_axis_name="c", subcore_axis_name="s",
        num_cores=NCORES, num_subcores=NSUB)


def _zero_vbuf(z0):
    def body(i, _):
        z0[i, pl.ds(0, 16)] = jnp.zeros((16,), jnp.float32)
        z0[i, pl.ds(16, 16)] = jnp.zeros((16,), jnp.float32)
        return 0
    lax.fori_loop(0, 128, body, 0)


def _scsum_body(with_cnt_outs, t0, t1, t2, t3, colp, rowp, coln, rown,
                *rest):
    if with_cnt_outs:
        (op0, op1, op2, op3, on0, on1, on2, on3, ocntp, ocntn,
         colbuf, rowbuf, rows_v, z0, z0c, ones_v, cacc, acc,
         gsem, ssem, isem, zsem) = rest
    else:
        (op0, op1, op2, op3, on0, on1, on2, on3,
         colbuf, rowbuf, rows_v, z0, z0c, ones_v, cacc, acc,
         gsem, ssem, isem, zsem) = rest
        ocntp = ocntn = None
    c = lax.axis_index("c")
    s = lax.axis_index("s")

    _zero_vbuf(z0)
    for i in range(8):
        z0c[pl.ds(i * 16, 16)] = jnp.zeros((16,), jnp.float32)
        ones_v[pl.ds(i * 16, 16)] = jnp.ones((16,), jnp.float32)

    def one_pass(t_ref, col2d, row2d, o_ref, ocnt, with_cnt):
        base = s * TROW
        def zb(i, _):
            pltpu.async_copy(z0, acc.at[pl.ds(base + i * 128, 128)], zsem)
            return 0
        lax.fori_loop(0, 24, zb, 0)
        pltpu.async_copy(z0.at[pl.ds(0, 64)],
                         acc.at[pl.ds(base + 24 * 128, 64)], zsem)
        def zbw(i, _):
            pltpu.make_async_copy(
                z0, acc.at[pl.ds(base + i * 128, 128)], zsem).wait()
            return 0
        if with_cnt:
            @pl.when(c == 0)
            def _():
                def zc(i, _):
                    pltpu.sync_copy(z0c, cacc.at[pl.ds(base + i * 128, 128)])
                    return 0
                lax.fori_loop(0, 24, zc, 0)
                pltpu.sync_copy(z0c.at[pl.ds(0, 64)],
                                cacc.at[pl.ds(base + 24 * 128, 64)])
        lax.fori_loop(0, 24, zbw, 0)
        pltpu.make_async_copy(
            z0.at[pl.ds(0, 64)],
            acc.at[pl.ds(base + 24 * 128, 64)], zsem).wait()
        plsc.subcore_barrier()

        ebase = s * (NW_T * 128)

        def stage_idx(w):
            b8 = lax.rem(w, 8)
            pltpu.async_copy(col2d.at[pl.ds(ebase + w * 128, 128)],
                             colbuf.at[b8], isem.at[b8])
            pltpu.async_copy(row2d.at[pl.ds(ebase + w * 128, 128)],
                             rowbuf.at[b8], isem.at[b8])

        def wait_idx(w):
            b8 = lax.rem(w, 8)
            pltpu.make_async_copy(
                col2d.at[pl.ds(ebase + w * 128, 128)],
                colbuf.at[b8], isem.at[b8]).wait()
            pltpu.make_async_copy(
                row2d.at[pl.ds(ebase + w * 128, 128)],
                rowbuf.at[b8], isem.at[b8]).wait()

        for w0 in (0, 1, 2):
            stage_idx(w0)
        for w0 in (0, 1):
            wait_idx(w0)
            pltpu.async_copy(t_ref.at[colbuf.at[w0]], rows_v.at[w0],
                             gsem.at[w0])

        def win(w, _):
            b = lax.rem(w, 4)
            b8 = lax.rem(w, 8)
            bg = lax.rem(w + 2, 4)
            bg8 = lax.rem(w + 2, 8)

            @pl.when(w >= 2)
            def _():
                pltpu.make_async_copy(
                    rows_v.at[bg], acc.at[rowbuf.at[lax.rem(w - 2, 8)]],
                    ssem.at[bg]).wait()

            @pl.when(w + 3 < NW_T)
            def _():
                stage_idx(w + 3)

            @pl.when(w + 2 < NW_T)
            def _():
                wait_idx(w + 2)
                pltpu.async_copy(t_ref.at[colbuf.at[bg8]], rows_v.at[bg],
                                 gsem.at[bg])

            pltpu.make_async_copy(
                t_ref.at[colbuf.at[b8]], rows_v.at[b], gsem.at[b]).wait()
            pltpu.async_copy(rows_v.at[b], acc.at[rowbuf.at[b8]],
                             ssem.at[b], add=True)
            if with_cnt:
                @pl.when(c == 0)
                def _():
                    pltpu.sync_copy(ones_v, cacc.at[rowbuf.at[b8]], add=True)
            return 0
        lax.fori_loop(0, NW_T, win, 0)
        for w in (NW_T - 2, NW_T - 1):
            pltpu.make_async_copy(
                rows_v.at[w % 4], acc.at[rowbuf.at[w % 8]],
                ssem.at[w % 4]).wait()
        plsc.subcore_barrier()

        pltpu.sync_copy(acc.at[pl.ds(base, TROW)], o_ref.at[pl.ds(base, TROW)])
        if with_cnt:
            @pl.when(c == 0)
            def _():
                pltpu.sync_copy(cacc.at[pl.ds(base, TROW)],
                                ocnt.at[pl.ds(base, TROW)])
        plsc.subcore_barrier()

    wc = with_cnt_outs

    @pl.when(c == 0)
    def _():
        one_pass(t0, colp, rowp, op0, ocntp, wc)
        one_pass(t0, coln, rown, on0, ocntn, wc)
        one_pass(t1, colp, rowp, op1, None, False)
        one_pass(t1, coln, rown, on1, None, False)

    @pl.when(c == 1)
    def _():
        one_pass(t2, colp, rowp, op2, None, False)
        one_pass(t2, coln, rown, on2, None, False)
        one_pass(t3, colp, rowp, op3, None, False)
        one_pass(t3, coln, rown, on3, None, False)


def _scsum(t0, t1, t2, t3, colp, rowp, coln, rown, with_cnt):
    f32 = jnp.float32
    sums = [jax.ShapeDtypeStruct((NACC, 32), f32)] * 8
    cnts = [jax.ShapeDtypeStruct((NACC,), f32)] * 2 if with_cnt else []
    kfn = pl.kernel(
        functools.partial(_scsum_body, with_cnt),
        out_type=tuple(sums + cnts),
        mesh=_vsmesh(),
        scratch_types=[
            pltpu.VMEM((8, 128), jnp.int32),
            pltpu.VMEM((8, 128), jnp.int32),
            pltpu.VMEM((4, 128, 32), f32),
            pltpu.VMEM((128, 32), f32),
            pltpu.VMEM((128,), f32),
            pltpu.VMEM((128,), f32),
            pltpu.VMEM_SHARED((NACC,), f32),
            pltpu.VMEM_SHARED((NACC, 32), f32),
            pltpu.SemaphoreType.DMA((4,)),
            pltpu.SemaphoreType.DMA((4,)),
            pltpu.SemaphoreType.DMA((8,)),
            pltpu.SemaphoreType.DMA,
        ],
        compiler_params=pltpu.CompilerParams(use_tc_tiling_on_sc=False),
    )
    return kfn(t0, t1, t2, t3, colp, rowp, coln, rown)


def _loss_body(z_hbm, p_hbm, ip, jp, kp, inn, jn, kn,
               ologits_p, ologits_n, oterms,
               ibuf, jbuf, kbuf, zi, zj, zk, pi, pj, pk, logbuf, tbuf, gsem):
    c = lax.axis_index("c")
    s = lax.axis_index("s")
    wid = s * NCORES + c

    lanes = lax.iota(jnp.int32, 16)

    def run_set(i2d, j2d, k2d, ologits, trow):
        base_w = wid * NW_L

        def stage(w, b):
            eb = wid * (NW_L * 128)
            pltpu.sync_copy(i2d.at[pl.ds(eb + w * 128, 128)], ibuf.at[b])
            pltpu.sync_copy(j2d.at[pl.ds(eb + w * 128, 128)], jbuf.at[b])
            pltpu.sync_copy(k2d.at[pl.ds(eb + w * 128, 128)], kbuf.at[b])
            pltpu.async_copy(z_hbm.at[ibuf.at[b]], zi.at[b], gsem.at[b])
            pltpu.async_copy(z_hbm.at[jbuf.at[b]], zj.at[b], gsem.at[b])
            pltpu.async_copy(z_hbm.at[kbuf.at[b]], zk.at[b], gsem.at[b])
            pltpu.async_copy(p_hbm.at[ibuf.at[b]], pi.at[b], gsem.at[b])
            pltpu.async_copy(p_hbm.at[jbuf.at[b]], pj.at[b], gsem.at[b])
            pltpu.async_copy(p_hbm.at[kbuf.at[b]], pk.at[b], gsem.at[b])

        def drain(b):
            pltpu.make_async_copy(z_hbm.at[ibuf.at[b]], zi.at[b],
                                  gsem.at[b]).wait()
            pltpu.make_async_copy(z_hbm.at[jbuf.at[b]], zj.at[b],
                                  gsem.at[b]).wait()
            pltpu.make_async_copy(z_hbm.at[kbuf.at[b]], zk.at[b],
                                  gsem.at[b]).wait()
            pltpu.make_async_copy(p_hbm.at[ibuf.at[b]], pi.at[b],
                                  gsem.at[b]).wait()
            pltpu.make_async_copy(p_hbm.at[jbuf.at[b]], pj.at[b],
                                  gsem.at[b]).wait()
            pltpu.make_async_copy(p_hbm.at[kbuf.at[b]], pk.at[b],
                                  gsem.at[b]).wait()

        stage(0, 0)

        def win(w, tacc):
            b = lax.rem(w, 2)
            bn = lax.rem(w + 1, 2)

            @pl.when(w + 1 < NW_L)
            def _():
                stage(w + 1, bn)

            drain(b)
            zib, zjb, zkb = zi.at[b], zj.at[b], zk.at[b]
            pib, pjb, pkb = pi.at[b], pj.at[b], pk.at[b]
            lgb = logbuf.at[b]

            rot = lanes * 5
            himask = jnp.full((16,), -65536, jnp.int32)
            for g in range(8):
                rowv = lanes + (g * 16)

                def feat(f, accs):
                    a, a2 = accs
                    fv = jnp.bitwise_and(rot + f, 63)
                    vi = plsc.load_gather(zib, [rowv, fv])
                    vj = plsc.load_gather(zjb, [rowv, fv])
                    vk = plsc.load_gather(zkb, [rowv, fv])
                    ilo = plsc.bitcast(vi << 16, jnp.float32)
                    jlo = plsc.bitcast(vj << 16, jnp.float32)
                    klo = plsc.bitcast(vk << 16, jnp.float32)
                    ihi = plsc.bitcast(jnp.bitwise_and(vi, himask),
                                       jnp.float32)
                    jhi = plsc.bitcast(jnp.bitwise_and(vj, himask),
                                       jnp.float32)
                    khi = plsc.bitcast(jnp.bitwise_and(vk, himask),
                                       jnp.float32)
                    return (a + jlo * (klo - ilo), a2 + jhi * (khi - ihi))
                z16 = jnp.zeros((16,), jnp.float32)
                a, a2 = lax.fori_loop(0, 64, feat, (z16, z16), unroll=8)
                a = a + a2

                c0 = jnp.zeros((16,), jnp.int32)
                sqi = plsc.load_gather(pib, [rowv, c0 + 4])
                sqk = plsc.load_gather(pkb, [rowv, c0 + 4])
                u0 = plsc.load_gather(pib, [rowv, c0])
                u1 = plsc.load_gather(pib, [rowv, c0 + 1])
                v0 = plsc.load_gather(pjb, [rowv, c0 + 2])
                v1 = plsc.load_gather(pjb, [rowv, c0 + 3])
                term = jnp.maximum(sqi - sqk + 2.0 * a, 0.0)
                tacc = tacc + term
                rr = jnp.zeros((16,), jnp.int32) + (g // 4)
                cc = jnp.bitwise_and(rowv * 2, 127)
                plsc.store_scatter(lgb, [rr, cc], u0 + v0)
                plsc.store_scatter(lgb, [rr, cc + 1], u1 + v1)

            pltpu.sync_copy(lgb,
                            ologits.at[pl.ds((base_w + w) * 2, 2)])
            return tacc

        tacc = lax.fori_loop(0, NW_L, win, jnp.zeros((16,), jnp.float32))
        tbuf[0, pl.ds(0, 16)] = tacc
        pltpu.sync_copy(tbuf, oterms.at[pl.ds(trow + wid, 1)])

    run_set(ip, jp, kp, ologits_p, 0)
    run_set(inn, jn, kn, ologits_n, 32)


def _loss_gather(z, p, ip, jp, kp, inn, jn, kn):
    f32 = jnp.float32
    kfn = pl.kernel(
        _loss_body,
        out_type=(
            jax.ShapeDtypeStruct((NLR, 128), f32),
            jax.ShapeDtypeStruct((NLR, 128), f32),
            jax.ShapeDtypeStruct((64, 16), f32),
        ),
        mesh=_vsmesh(),
        scratch_types=[
            pltpu.VMEM((2, 128), jnp.int32),
            pltpu.VMEM((2, 128), jnp.int32),
            pltpu.VMEM((2, 128), jnp.int32),
            pltpu.VMEM((2, 128, 64), jnp.int32),
            pltpu.VMEM((2, 128, 64), jnp.int32),
            pltpu.VMEM((2, 128, 64), jnp.int32),
            pltpu.VMEM((2, 128, 16), f32),
            pltpu.VMEM((2, 128, 16), f32),
            pltpu.VMEM((2, 128, 16), f32),
            pltpu.VMEM((2, 2, 128), f32),
            pltpu.VMEM((1, 16), f32),
            pltpu.SemaphoreType.DMA((2,)),
        ],
        compiler_params=pltpu.CompilerParams(
            use_tc_tiling_on_sc=False, needs_layout_passes=False),
    )
    return kfn(z, p, ip, jp, kp, inn, jn, kn)


def _l2n(x):
    return x / jnp.maximum(
        jnp.sqrt(jnp.sum(x * x, axis=-1, keepdims=True)), 1e-12)


def _layer1_body(x0, x1, x2, x3, sp0, sp1, sp2, sp3, sn0, sn1, sn2, sn3,
                 cp, cn, wp, bp, wn, bn, h0, h1, h2, h3):
    x = jnp.concatenate([x0[...], x1[...], x2[...], x3[...]], axis=1)
    aggp = jnp.concatenate([sp0[...], sp1[...], sp2[...], sp3[...]], axis=1)
    aggn = jnp.concatenate([sn0[...], sn1[...], sn2[...], sn3[...]], axis=1)
    aggp = aggp / jnp.maximum(cp[...], 1.0)
    aggn = aggn / jnp.maximum(cn[...], 1.0)
    hp = jnp.tanh(_l2n(
        jnp.concatenate([aggp, x], axis=1) @ wp[...] + bp[...]))
    hn = jnp.tanh(_l2n(
        jnp.concatenate([aggn, x], axis=1) @ wn[...] + bn[...]))
    h0[...] = hp[:, 0:32]
    h1[...] = hp[:, 32:64]
    h2[...] = hn[:, 0:32]
    h3[...] = hn[:, 32:64]


def _layer1(xq, sp, sn, cp2, cn2, wp, bp, wn, bn):
    f32 = jnp.float32
    bspec = pl.BlockSpec((BN, 32), lambda i: (i, 0))
    cspec = pl.BlockSpec((BN, 1), lambda i: (i, 0))
    full = lambda shp: pl.BlockSpec(shp, lambda i: (0, 0))
    return pl.pallas_call(
        _layer1_body,
        grid=(NACC // BN,),
        in_specs=[bspec] * 12 + [cspec, cspec,
                                 full((2 * D, H)), full((1, H)),
                                 full((2 * D, H)), full((1, H))],
        out_specs=[bspec] * 4,
        out_shape=[jax.ShapeDtypeStruct((NACC, 32), f32)] * 4,
    )(*xq, *sp, *sn, cp2, cn2, wp, bp, wn, bn)


def _layer2_body(h0, h1, h2, h3, pe0, pe1, pe2, pe3, ne0, ne1, ne2, ne3,
                 cp, cn, wp, bp, wn, bn, wr, zo, po):
    hp = jnp.concatenate([h0[...], h1[...]], axis=1)
    hn = jnp.concatenate([h2[...], h3[...]], axis=1)
    icp = 1.0 / jnp.maximum(cp[...], 1.0)
    icn = 1.0 / jnp.maximum(cn[...], 1.0)
    out1 = jnp.concatenate([pe0[...], pe1[...]], axis=1) * icp
    out1n = jnp.concatenate([pe2[...], pe3[...]], axis=1) * icp
    out2n = jnp.concatenate([ne0[...], ne1[...]], axis=1) * icn
    out2 = jnp.concatenate([ne2[...], ne3[...]], axis=1) * icn
    hp2 = jnp.tanh(_l2n(
        jnp.concatenate([out1, out2, hp], axis=1) @ wp[...] + bp[...]))
    hn2 = jnp.tanh(_l2n(
        jnp.concatenate([out1n, out2n, hn], axis=1) @ wn[...] + bn[...]))
    z = jnp.concatenate([hp2, hn2], axis=1)
    zo[...] = z
    wr_full = wr[...]
    u = z @ wr_full[0:128, :]
    v = z @ wr_full[128:256, :]
    sq = jnp.sum(z * z, axis=1, keepdims=True)
    po[...] = jnp.concatenate(
        [u, v, sq, jnp.zeros((z.shape[0], 11), jnp.float32)], axis=1)


def _layer2(hq, spe, sne, cp2, cn2, wp, bp, wn, bn, wr):
    f32 = jnp.float32
    bspec = pl.BlockSpec((BN, 32), lambda i: (i, 0))
    cspec = pl.BlockSpec((BN, 1), lambda i: (i, 0))
    full = lambda shp: pl.BlockSpec(shp, lambda i: (0, 0))
    return pl.pallas_call(
        _layer2_body,
        grid=(NACC // BN,),
        in_specs=[bspec] * 12 + [cspec, cspec,
                                 full((3 * H, H)), full((1, H)),
                                 full((3 * H, H)), full((1, H)),
                                 full((4 * H, 2))],
        out_specs=[pl.BlockSpec((BN, D), lambda i: (i, 0)),
                   pl.BlockSpec((BN, 16), lambda i: (i, 0))],
        out_shape=[jax.ShapeDtypeStruct((NACC, D), f32),
                   jax.ShapeDtypeStruct((NACC, 16), f32)],
    )(*hq, *spe, *sne, cp2, cn2, wp, bp, wn, bn, wr)


NLR = EPAD * 2 // 128
BE = NLR // 7


def _nll_body(lp, tp, ln, tn, o):
    pid = pl.program_id(0)

    def one(lg, tg):
        x = lg[...]
        t = jnp.repeat(tg[...], 2, axis=1)
        b = jnp.roll(x, -1, axis=1)
        m = jnp.maximum(x, b)
        lse = m + jnp.log(jnp.exp(x - m) + jnp.exp(b - m))
        lt = jnp.where(t == 0, x, b)
        lane = jax.lax.broadcasted_iota(jnp.int32, (BE, 128), 1)
        row = jax.lax.broadcasted_iota(jnp.int32, (BE, 128), 0) + pid * BE
        edge = row * 64 + lane // 2
        mask = (jnp.bitwise_and(lane, 1) == 0) & (edge < E_POS)
        return jnp.sum(jnp.where(mask, lse - lt, 0.0))

    @pl.when(pid == 0)
    def _():
        o[...] = jnp.zeros_like(o)
    o[...] += (one(lp, tp) + one(ln, tn)).reshape(1, 1)


def _nll_sum(logits_p, tp2, logits_n, tn2):
    bspec = pl.BlockSpec((BE, 128), lambda i: (i, 0))
    tspec = pl.BlockSpec((BE, 64), lambda i: (i, 0))
    return pl.pallas_call(
        _nll_body,
        grid=(7,),
        in_specs=[bspec, tspec, bspec, tspec],
        out_specs=pl.BlockSpec((1, 1), lambda i: (0, 0)),
        out_shape=jax.ShapeDtypeStruct((1, 1), jnp.float32),
    )(logits_p, tp2, logits_n, tn2)


def _pad_edges(col, row):
    pad = EPAD - col.shape[0]
    padcol = (jnp.arange(pad, dtype=jnp.int32) * 97) % N
    padrow = N + (jnp.arange(pad, dtype=jnp.int32) % 128)
    return jnp.concatenate([col, padcol]), jnp.concatenate([row, padrow])


def _pad_idx(a):
    pad = EPAD - a.shape[0]
    return jnp.concatenate([a, jnp.zeros((pad,), jnp.int32)])


def kernel(positive_edges, negative_edges, target, pos_samples, neg_samples,
           X, W_pos1, b_pos1, W_neg1, b_neg1, W_pos2, b_pos2, W_neg2, b_neg2,
           W_reg):
    rp, cp = positive_edges[0], positive_edges[1]
    rn, cn = negative_edges[0], negative_edges[1]

    colp2d, rowp3d = _pad_edges(cp, rp)
    coln2d, rown3d = _pad_edges(cn, rn)

    Xp = jnp.pad(X, ((0, NACC - N), (0, 0)))
    xq = tuple(Xp[:, q * 32:(q + 1) * 32] for q in range(4))

    *sums1, cntp, cntn = _scsum(*xq, colp2d, rowp3d, coln2d, rown3d, True)
    sp, sn = sums1[0:4], sums1[4:8]
    cntp = cntp.reshape(NACC, 1)
    cntn = cntn.reshape(NACC, 1)

    hq = _layer1(xq, sp, sn, cntp, cntn,
                 W_pos1, b_pos1.reshape(1, H), W_neg1, b_neg1.reshape(1, H))

    sums2 = _scsum(*hq, colp2d, rowp3d, coln2d, rown3d, False)
    spe, sne = sums2[0:4], sums2[4:8]

    z, ptab = _layer2(hq, spe, sne, cntp, cntn,
                      W_pos2, b_pos2.reshape(1, H),
                      W_neg2, b_neg2.reshape(1, H), W_reg)

    ip2d, jp2d, kp2d = _pad_idx(rp), _pad_idx(cp), _pad_idx(pos_samples)
    in2d, jn2d, kn2d = _pad_idx(rn), _pad_idx(cn), _pad_idx(neg_samples)

    zpk = jax.lax.bitcast_convert_type(
        z.astype(jnp.bfloat16).reshape(NACC, 64, 2), jnp.int32)
    logits_p, logits_n, terms = _loss_gather(
        zpk, ptab, ip2d, jp2d, kp2d, in2d, jn2d, kn2d)
    terms_p, terms_n = terms[:32], terms[32:]

    tp2 = jnp.concatenate(
        [target[:E_POS], jnp.zeros((EPAD - E_POS,), jnp.int32)]
    ).reshape(NLR, 64)
    tn2 = jnp.concatenate(
        [target[E_POS:], jnp.zeros((EPAD - E_NEG,), jnp.int32)]
    ).reshape(NLR, 64)
    nll = _nll_sum(logits_p, tp2,
                   logits_n, tn2)[0, 0] / (E_POS + E_NEG)

    loss_p = jnp.sum(terms_p) / E_POS
    loss_n = jnp.sum(terms_n) / E_NEG
    loss = nll + LAMB * (loss_p + loss_n)
    return (loss, z[:N])


# speedup vs baseline: 1.6994x; 1.0000x over previous
import functools

import jax
import jax.numpy as jnp
from jax import lax
from jax.experimental import pallas as pl
from jax.experimental.pallas import tpu as pltpu
from jax.experimental.pallas import tpu_sc as plsc

N = 50000
D = 128
H = 64
E_POS = 400000
E_NEG = 400000
LAMB = 1.0

BN = 512
NACC = 50176
TROW = NACC // 16
EPAD = 401408
NWIN = EPAD // 128
NW_T = NWIN // 16
NW_L = NWIN // 32
NCORES = 2
NSUB = 16


def _vsmesh():
    return plsc.VectorSubcoreMesh(
        core_axis_name="c", subcore_axis_name="s",
        num_cores=NCORES, num_subcores=NSUB)


def _zero_vbuf(z0):
    def body(i, _):
        z0[i, pl.ds(0, 16)] = jnp.zeros((16,), jnp.float32)
        z0[i, pl.ds(16, 16)] = jnp.zeros((16,), jnp.float32)
        return 0
    lax.fori_loop(0, 128, body, 0)


def _scsum_body(with_cnt_outs, t0, t1, t2, t3, colp, rowp, coln, rown,
                *rest):
    if with_cnt_outs:
        (op0, op1, op2, op3, on0, on1, on2, on3, ocntp, ocntn,
         colbuf, rowbuf, rows_v, z0, z0c, ones_v, cacc, acc,
         gsem, ssem, isem, zsem) = rest
    else:
        (op0, op1, op2, op3, on0, on1, on2, on3,
         colbuf, rowbuf, rows_v, z0, z0c, ones_v, cacc, acc,
         gsem, ssem, isem, zsem) = rest
        ocntp = ocntn = None
    c = lax.axis_index("c")
    s = lax.axis_index("s")

    _zero_vbuf(z0)
    for i in range(8):
        z0c[pl.ds(i * 16, 16)] = jnp.zeros((16,), jnp.float32)
        ones_v[pl.ds(i * 16, 16)] = jnp.ones((16,), jnp.float32)

    def one_pass(t_ref, col2d, row2d, o_ref, ocnt, with_cnt):
        base = s * TROW
        def zb(i, _):
            pltpu.async_copy(z0, acc.at[pl.ds(base + i * 128, 128)], zsem)
            return 0
        lax.fori_loop(0, 24, zb, 0)
        pltpu.async_copy(z0.at[pl.ds(0, 64)],
                         acc.at[pl.ds(base + 24 * 128, 64)], zsem)
        def zbw(i, _):
            pltpu.make_async_copy(
                z0, acc.at[pl.ds(base + i * 128, 128)], zsem).wait()
            return 0
        if with_cnt:
            @pl.when(c == 0)
            def _():
                def zc(i, _):
                    pltpu.sync_copy(z0c, cacc.at[pl.ds(base + i * 128, 128)])
                    return 0
                lax.fori_loop(0, 24, zc, 0)
                pltpu.sync_copy(z0c.at[pl.ds(0, 64)],
                                cacc.at[pl.ds(base + 24 * 128, 64)])
        lax.fori_loop(0, 24, zbw, 0)
        pltpu.make_async_copy(
            z0.at[pl.ds(0, 64)],
            acc.at[pl.ds(base + 24 * 128, 64)], zsem).wait()
        plsc.subcore_barrier()

        ebase = s * (NW_T * 128)

        def stage_idx(w):
            b8 = lax.rem(w, 8)
            pltpu.async_copy(col2d.at[pl.ds(ebase + w * 128, 128)],
                             colbuf.at[b8], isem.at[b8])
            pltpu.async_copy(row2d.at[pl.ds(ebase + w * 128, 128)],
                             rowbuf.at[b8], isem.at[b8])

        def wait_idx(w):
            b8 = lax.rem(w, 8)
            pltpu.make_async_copy(
                col2d.at[pl.ds(ebase + w * 128, 128)],
                colbuf.at[b8], isem.at[b8]).wait()
            pltpu.make_async_copy(
                row2d.at[pl.ds(ebase + w * 128, 128)],
                rowbuf.at[b8], isem.at[b8]).wait()

        for w0 in (0, 1, 2):
            stage_idx(w0)
        for w0 in (0, 1):
            wait_idx(w0)
            pltpu.async_copy(t_ref.at[colbuf.at[w0]], rows_v.at[w0],
                             gsem.at[w0])

        def win(w, _):
            b = lax.rem(w, 4)
            b8 = lax.rem(w, 8)
            bg = lax.rem(w + 2, 4)
            bg8 = lax.rem(w + 2, 8)

            @pl.when(w >= 2)
            def _():
                pltpu.make_async_copy(
                    rows_v.at[bg], acc.at[rowbuf.at[lax.rem(w - 2, 8)]],
                    ssem.at[bg]).wait()

            @pl.when(w + 3 < NW_T)
            def _():
                stage_idx(w + 3)

            @pl.when(w + 2 < NW_T)
            def _():
                wait_idx(w + 2)
                pltpu.async_copy(t_ref.at[colbuf.at[bg8]], rows_v.at[bg],
                                 gsem.at[bg])

            pltpu.make_async_copy(
                t_ref.at[colbuf.at[b8]], rows_v.at[b], gsem.at[b]).wait()
            pltpu.async_copy(rows_v.at[b], acc.at[rowbuf.at[b8]],
                             ssem.at[b], add=True)
            if with_cnt:
                @pl.when(c == 0)
                def _():
                    pltpu.sync_copy(ones_v, cacc.at[rowbuf.at[b8]], add=True)
            return 0
        lax.fori_loop(0, NW_T, win, 0)
        for w in (NW_T - 2, NW_T - 1):
            pltpu.make_async_copy(
                rows_v.at[w % 4], acc.at[rowbuf.at[w % 8]],
                ssem.at[w % 4]).wait()
        plsc.subcore_barrier()

        pltpu.sync_copy(acc.at[pl.ds(base, TROW)], o_ref.at[pl.ds(base, TROW)])
        if with_cnt:
            @pl.when(c == 0)
            def _():
                pltpu.sync_copy(cacc.at[pl.ds(base, TROW)],
                                ocnt.at[pl.ds(base, TROW)])
        plsc.subcore_barrier()

    wc = with_cnt_outs

    @pl.when(c == 0)
    def _():
        one_pass(t0, colp, rowp, op0, ocntp, wc)
        one_pass(t0, coln, rown, on0, ocntn, wc)
        one_pass(t1, colp, rowp, op1, None, False)
        one_pass(t1, coln, rown, on1, None, False)

    @pl.when(c == 1)
    def _():
        one_pass(t2, colp, rowp, op2, None, False)
        one_pass(t2, coln, rown, on2, None, False)
        one_pass(t3, colp, rowp, op3, None, False)
        one_pass(t3, coln, rown, on3, None, False)


def _scsum(t0, t1, t2, t3, colp, rowp, coln, rown, with_cnt):
    f32 = jnp.float32
    sums = [jax.ShapeDtypeStruct((NACC, 32), f32)] * 8
    cnts = [jax.ShapeDtypeStruct((NACC,), f32)] * 2 if with_cnt else []
    kfn = pl.kernel(
        functools.partial(_scsum_body, with_cnt),
        out_type=tuple(sums + cnts),
        mesh=_vsmesh(),
        scratch_types=[
            pltpu.VMEM((8, 128), jnp.int32),
            pltpu.VMEM((8, 128), jnp.int32),
            pltpu.VMEM((4, 128, 32), f32),
            pltpu.VMEM((128, 32), f32),
            pltpu.VMEM((128,), f32),
            pltpu.VMEM((128,), f32),
            pltpu.VMEM_SHARED((NACC,), f32),
            pltpu.VMEM_SHARED((NACC, 32), f32),
            pltpu.SemaphoreType.DMA((4,)),
            pltpu.SemaphoreType.DMA((4,)),
            pltpu.SemaphoreType.DMA((8,)),
            pltpu.SemaphoreType.DMA,
        ],
        compiler_params=pltpu.CompilerParams(use_tc_tiling_on_sc=False),
    )
    return kfn(t0, t1, t2, t3, colp, rowp, coln, rown)


def _loss_body(z_hbm, p_hbm, ip, jp, kp, inn, jn, kn,
               ologits_p, ologits_n, oterms,
               ibuf, jbuf, kbuf, zi, zj, zk, pi, pj, pk, logbuf, tbuf, gsem):
    c = lax.axis_index("c")
    s = lax.axis_index("s")
    wid = s * NCORES + c

    lanes = lax.iota(jnp.int32, 16)

    def run_set(i2d, j2d, k2d, ologits, trow):
        base_w = wid * NW_L

        def stage(w, b):
            eb = wid * (NW_L * 128)
            pltpu.sync_copy(i2d.at[pl.ds(eb + w * 128, 128)], ibuf.at[b])
            pltpu.sync_copy(j2d.at[pl.ds(eb + w * 128, 128)], jbuf.at[b])
            pltpu.sync_copy(k2d.at[pl.ds(eb + w * 128, 128)], kbuf.at[b])
            pltpu.async_copy(z_hbm.at[ibuf.at[b]], zi.at[b], gsem.at[b])
            pltpu.async_copy(z_hbm.at[jbuf.at[b]], zj.at[b], gsem.at[b])
            pltpu.async_copy(z_hbm.at[kbuf.at[b]], zk.at[b], gsem.at[b])
            pltpu.async_copy(p_hbm.at[ibuf.at[b]], pi.at[b], gsem.at[b])
            pltpu.async_copy(p_hbm.at[jbuf.at[b]], pj.at[b], gsem.at[b])
            pltpu.async_copy(p_hbm.at[kbuf.at[b]], pk.at[b], gsem.at[b])

        def drain(b):
            pltpu.make_async_copy(z_hbm.at[ibuf.at[b]], zi.at[b],
                                  gsem.at[b]).wait()
            pltpu.make_async_copy(z_hbm.at[jbuf.at[b]], zj.at[b],
                                  gsem.at[b]).wait()
            pltpu.make_async_copy(z_hbm.at[kbuf.at[b]], zk.at[b],
                                  gsem.at[b]).wait()
            pltpu.make_async_copy(p_hbm.at[ibuf.at[b]], pi.at[b],
                                  gsem.at[b]).wait()
            pltpu.make_async_copy(p_hbm.at[jbuf.at[b]], pj.at[b],
                                  gsem.at[b]).wait()
            pltpu.make_async_copy(p_hbm.at[kbuf.at[b]], pk.at[b],
                                  gsem.at[b]).wait()

        stage(0, 0)
        stage(1, 1)

        def win(w, tacc):
            b = lax.rem(w, 3)

            @pl.when(w + 2 < NW_L)
            def _():
                stage(w + 2, lax.rem(w + 2, 3))

            drain(b)
            zib, zjb, zkb = zi.at[b], zj.at[b], zk.at[b]
            pib, pjb, pkb = pi.at[b], pj.at[b], pk.at[b]
            lgb = logbuf.at[lax.rem(w, 2)]

            rot = lanes * 5
            himask = jnp.full((16,), -65536, jnp.int32)
            for g in range(8):
                rowv = lanes + (g * 16)

                def feat(f, accs):
                    a, a2 = accs
                    fv = jnp.bitwise_and(rot + f, 63)
                    vi = plsc.load_gather(zib, [rowv, fv])
                    vj = plsc.load_gather(zjb, [rowv, fv])
                    vk = plsc.load_gather(zkb, [rowv, fv])
                    ilo = plsc.bitcast(vi << 16, jnp.float32)
                    jlo = plsc.bitcast(vj << 16, jnp.float32)
                    klo = plsc.bitcast(vk << 16, jnp.float32)
                    ihi = plsc.bitcast(jnp.bitwise_and(vi, himask),
                                       jnp.float32)
                    jhi = plsc.bitcast(jnp.bitwise_and(vj, himask),
                                       jnp.float32)
                    khi = plsc.bitcast(jnp.bitwise_and(vk, himask),
                                       jnp.float32)
                    return (a + jlo * (klo - ilo), a2 + jhi * (khi - ihi))
                z16 = jnp.zeros((16,), jnp.float32)
                a, a2 = lax.fori_loop(0, 64, feat, (z16, z16), unroll=8)
                a = a + a2

                c0 = jnp.zeros((16,), jnp.int32)
                sqi = plsc.load_gather(pib, [rowv, c0 + 4])
                sqk = plsc.load_gather(pkb, [rowv, c0 + 4])
                u0 = plsc.load_gather(pib, [rowv, c0])
                u1 = plsc.load_gather(pib, [rowv, c0 + 1])
                v0 = plsc.load_gather(pjb, [rowv, c0 + 2])
                v1 = plsc.load_gather(pjb, [rowv, c0 + 3])
                term = jnp.maximum(sqi - sqk + 2.0 * a, 0.0)
                tacc = tacc + term
                rr = jnp.zeros((16,), jnp.int32) + (g // 4)
                cc = jnp.bitwise_and(rowv * 2, 127)
                plsc.store_scatter(lgb, [rr, cc], u0 + v0)
                plsc.store_scatter(lgb, [rr, cc + 1], u1 + v1)

            pltpu.sync_copy(lgb,
                            ologits.at[pl.ds((base_w + w) * 2, 2)])
            return tacc

        tacc = lax.fori_loop(0, NW_L, win, jnp.zeros((16,), jnp.float32))
        tbuf[0, pl.ds(0, 16)] = tacc
        pltpu.sync_copy(tbuf, oterms.at[pl.ds(trow + wid, 1)])

    run_set(ip, jp, kp, ologits_p, 0)
    run_set(inn, jn, kn, ologits_n, 32)


def _loss_gather(z, p, ip, jp, kp, inn, jn, kn):
    f32 = jnp.float32
    kfn = pl.kernel(
        _loss_body,
        out_type=(
            jax.ShapeDtypeStruct((NLR, 128), f32),
            jax.ShapeDtypeStruct((NLR, 128), f32),
            jax.ShapeDtypeStruct((64, 16), f32),
        ),
        mesh=_vsmesh(),
        scratch_types=[
            pltpu.VMEM((3, 128), jnp.int32),
            pltpu.VMEM((3, 128), jnp.int32),
            pltpu.VMEM((3, 128), jnp.int32),
            pltpu.VMEM((3, 128, 64), jnp.int32),
            pltpu.VMEM((3, 128, 64), jnp.int32),
            pltpu.VMEM((3, 128, 64), jnp.int32),
            pltpu.VMEM((3, 128, 16), f32),
            pltpu.VMEM((3, 128, 16), f32),
            pltpu.VMEM((3, 128, 16), f32),
            pltpu.VMEM((2, 2, 128), f32),
            pltpu.VMEM((1, 16), f32),
            pltpu.SemaphoreType.DMA((3,)),
        ],
        compiler_params=pltpu.CompilerParams(
            use_tc_tiling_on_sc=False, needs_layout_passes=False),
    )
    return kfn(z, p, ip, jp, kp, inn, jn, kn)


def _l2n(x):
    return x / jnp.maximum(
        jnp.sqrt(jnp.sum(x * x, axis=-1, keepdims=True)), 1e-12)


def _layer1_body(x0, x1, x2, x3, sp0, sp1, sp2, sp3, sn0, sn1, sn2, sn3,
                 cp, cn, wp, bp, wn, bn, h0, h1, h2, h3):
    x = jnp.concatenate([x0[...], x1[...], x2[...], x3[...]], axis=1)
    aggp = jnp.concatenate([sp0[...], sp1[...], sp2[...], sp3[...]], axis=1)
    aggn = jnp.concatenate([sn0[...], sn1[...], sn2[...], sn3[...]], axis=1)
    aggp = aggp / jnp.maximum(cp[...], 1.0)
    aggn = aggn / jnp.maximum(cn[...], 1.0)
    hp = jnp.tanh(_l2n(
        jnp.concatenate([aggp, x], axis=1) @ wp[...] + bp[...]))
    hn = jnp.tanh(_l2n(
        jnp.concatenate([aggn, x], axis=1) @ wn[...] + bn[...]))
    h0[...] = hp[:, 0:32]
    h1[...] = hp[:, 32:64]
    h2[...] = hn[:, 0:32]
    h3[...] = hn[:, 32:64]


def _layer1(xq, sp, sn, cp2, cn2, wp, bp, wn, bn):
    f32 = jnp.float32
    bspec = pl.BlockSpec((BN, 32), lambda i: (i, 0))
    cspec = pl.BlockSpec((BN, 1), lambda i: (i, 0))
    full = lambda shp: pl.BlockSpec(shp, lambda i: (0, 0))
    return pl.pallas_call(
        _layer1_body,
        grid=(NACC // BN,),
        in_specs=[bspec] * 12 + [cspec, cspec,
                                 full((2 * D, H)), full((1, H)),
                                 full((2 * D, H)), full((1, H))],
        out_specs=[bspec] * 4,
        out_shape=[jax.ShapeDtypeStruct((NACC, 32), f32)] * 4,
    )(*xq, *sp, *sn, cp2, cn2, wp, bp, wn, bn)


def _layer2_body(h0, h1, h2, h3, pe0, pe1, pe2, pe3, ne0, ne1, ne2, ne3,
                 cp, cn, wp, bp, wn, bn, wr, zo, po):
    hp = jnp.concatenate([h0[...], h1[...]], axis=1)
    hn = jnp.concatenate([h2[...], h3[...]], axis=1)
    icp = 1.0 / jnp.maximum(cp[...], 1.0)
    icn = 1.0 / jnp.maximum(cn[...], 1.0)
    out1 = jnp.concatenate([pe0[...], pe1[...]], axis=1) * icp
    out1n = jnp.concatenate([pe2[...], pe3[...]], axis=1) * icp
    out2n = jnp.concatenate([ne0[...], ne1[...]], axis=1) * icn
    out2 = jnp.concatenate([ne2[...], ne3[...]], axis=1) * icn
    hp2 = jnp.tanh(_l2n(
        jnp.concatenate([out1, out2, hp], axis=1) @ wp[...] + bp[...]))
    hn2 = jnp.tanh(_l2n(
        jnp.concatenate([out1n, out2n, hn], axis=1) @ wn[...] + bn[...]))
    z = jnp.concatenate([hp2, hn2], axis=1)
    zo[...] = z
    wr_full = wr[...]
    u = z @ wr_full[0:128, :]
    v = z @ wr_full[128:256, :]
    sq = jnp.sum(z * z, axis=1, keepdims=True)
    po[...] = jnp.concatenate(
        [u, v, sq, jnp.zeros((z.shape[0], 11), jnp.float32)], axis=1)


def _layer2(hq, spe, sne, cp2, cn2, wp, bp, wn, bn, wr):
    f32 = jnp.float32
    bspec = pl.BlockSpec((BN, 32), lambda i: (i, 0))
    cspec = pl.BlockSpec((BN, 1), lambda i: (i, 0))
    full = lambda shp: pl.BlockSpec(shp, lambda i: (0, 0))
    return pl.pallas_call(
        _layer2_body,
        grid=(NACC // BN,),
        in_specs=[bspec] * 12 + [cspec, cspec,
                                 full((3 * H, H)), full((1, H)),
                                 full((3 * H, H)), full((1, H)),
                                 full((4 * H, 2))],
        out_specs=[pl.BlockSpec((BN, D), lambda i: (i, 0)),
                   pl.BlockSpec((BN, 16), lambda i: (i, 0))],
        out_shape=[jax.ShapeDtypeStruct((NACC, D), f32),
                   jax.ShapeDtypeStruct((NACC, 16), f32)],
    )(*hq, *spe, *sne, cp2, cn2, wp, bp, wn, bn, wr)


NLR = EPAD * 2 // 128
BE = NLR // 7


def _nll_body(lp, tp, ln, tn, o):
    pid = pl.program_id(0)

    def one(lg, tg):
        x = lg[...]
        t = jnp.repeat(tg[...], 2, axis=1)
        b = jnp.roll(x, -1, axis=1)
        m = jnp.maximum(x, b)
        lse = m + jnp.log(jnp.exp(x - m) + jnp.exp(b - m))
        lt = jnp.where(t == 0, x, b)
        lane = jax.lax.broadcasted_iota(jnp.int32, (BE, 128), 1)
        row = jax.lax.broadcasted_iota(jnp.int32, (BE, 128), 0) + pid * BE
        edge = row * 64 + lane // 2
        mask = (jnp.bitwise_and(lane, 1) == 0) & (edge < E_POS)
        return jnp.sum(jnp.where(mask, lse - lt, 0.0))

    @pl.when(pid == 0)
    def _():
        o[...] = jnp.zeros_like(o)
    o[...] += (one(lp, tp) + one(ln, tn)).reshape(1, 1)


def _nll_sum(logits_p, tp2, logits_n, tn2):
    bspec = pl.BlockSpec((BE, 128), lambda i: (i, 0))
    tspec = pl.BlockSpec((BE, 64), lambda i: (i, 0))
    return pl.pallas_call(
        _nll_body,
        grid=(7,),
        in_specs=[bspec, tspec, bspec, tspec],
        out_specs=pl.BlockSpec((1, 1), lambda i: (0, 0)),
        out_shape=jax.ShapeDtypeStruct((1, 1), jnp.float32),
    )(logits_p, tp2, logits_n, tn2)


def _pad_edges(col, row):
    pad = EPAD - col.shape[0]
    padcol = (jnp.arange(pad, dtype=jnp.int32) * 97) % N
    padrow = N + (jnp.arange(pad, dtype=jnp.int32) % 128)
    return jnp.concatenate([col, padcol]), jnp.concatenate([row, padrow])


def _pad_idx(a):
    pad = EPAD - a.shape[0]
    return jnp.concatenate([a, jnp.zeros((pad,), jnp.int32)])


def kernel(positive_edges, negative_edges, target, pos_samples, neg_samples,
           X, W_pos1, b_pos1, W_neg1, b_neg1, W_pos2, b_pos2, W_neg2, b_neg2,
           W_reg):
    rp, cp = positive_edges[0], positive_edges[1]
    rn, cn = negative_edges[0], negative_edges[1]

    colp2d, rowp3d = _pad_edges(cp, rp)
    coln2d, rown3d = _pad_edges(cn, rn)

    Xp = jnp.pad(X, ((0, NACC - N), (0, 0)))
    xq = tuple(Xp[:, q * 32:(q + 1) * 32] for q in range(4))

    *sums1, cntp, cntn = _scsum(*xq, colp2d, rowp3d, coln2d, rown3d, True)
    sp, sn = sums1[0:4], sums1[4:8]
    cntp = cntp.reshape(NACC, 1)
    cntn = cntn.reshape(NACC, 1)

    hq = _layer1(xq, sp, sn, cntp, cntn,
                 W_pos1, b_pos1.reshape(1, H), W_neg1, b_neg1.reshape(1, H))

    sums2 = _scsum(*hq, colp2d, rowp3d, coln2d, rown3d, False)
    spe, sne = sums2[0:4], sums2[4:8]

    z, ptab = _layer2(hq, spe, sne, cntp, cntn,
                      W_pos2, b_pos2.reshape(1, H),
                      W_neg2, b_neg2.reshape(1, H), W_reg)

    ip2d, jp2d, kp2d = _pad_idx(rp), _pad_idx(cp), _pad_idx(pos_samples)
    in2d, jn2d, kn2d = _pad_idx(rn), _pad_idx(cn), _pad_idx(neg_samples)

    zpk = jax.lax.bitcast_convert_type(
        z.astype(jnp.bfloat16).reshape(NACC, 64, 2), jnp.int32)
    logits_p, logits_n, terms = _loss_gather(
        zpk, ptab, ip2d, jp2d, kp2d, in2d, jn2d, kn2d)
    terms_p, terms_n = terms[:32], terms[32:]

    tp2 = jnp.concatenate(
        [target[:E_POS], jnp.zeros((EPAD - E_POS,), jnp.int32)]
    ).reshape(NLR, 64)
    tn2 = jnp.concatenate(
        [target[E_POS:], jnp.zeros((EPAD - E_NEG,), jnp.int32)]
    ).reshape(NLR, 64)
    nll = _nll_sum(logits_p, tp2,
                   logits_n, tn2)[0, 0] / (E_POS + E_NEG)

    loss_p = jnp.sum(terms_p) / E_POS
    loss_n = jnp.sum(terms_n) / E_NEG
    loss = nll + LAMB * (loss_p + loss_n)
    return (loss, z[:N])
